# Initial kernel scaffold; baseline (speedup 1.0000x reference)
import jax
import jax.numpy as jnp
from jax.experimental import pallas as pl


def kernel(u, i, j, sgn, edge_index, emb_pos_u, emb_pos_i, emb_neg_u, emb_neg_i, W_mlp0, b_mlp0, W_mlp1, b_mlp1, W_attn, b_attn, W_q):
    raise NotImplementedError("write your pallas kernel here")


# speedup vs baseline: 8.9356x; 8.9356x over previous
import functools

import jax
import jax.numpy as jnp
from jax import lax
from jax.experimental import pallas as pl
from jax.experimental.pallas import tpu as pltpu
from jax.experimental.pallas import tpu_sc as plsc

_NU = 30000
_NV = 20000
_N = 50000
_D = 64
_PAD_U = 30720
_PAD_N = 51200
_ITEM_SHIFT = _PAD_U - _NU
_E_HALF = 400000
_NCHUNK = _E_HALF // 128
_GMAX = (_NCHUNK + 15) // 16
_B = 16384
_REG = 0.05
_GAMMA = 1e-10

_MESH = dict(core_axis_name="c", subcore_axis_name="s", num_cores=2,
             num_subcores=16)
_SC_PARAMS = pltpu.CompilerParams(use_tc_tiling_on_sc=False)


def _worker_ids():
  c = lax.axis_index("c")
  s = lax.axis_index("s")
  return c, s


@functools.partial(
    pl.kernel,
    out_type=[
        jax.ShapeDtypeStruct((2 * _E_HALF,), jnp.int32),
        jax.ShapeDtypeStruct((2 * _E_HALF,), jnp.int32),
    ],
    mesh=plsc.VectorSubcoreMesh(**_MESH),
    compiler_params=_SC_PARAMS,
    scratch_types=[
        pltpu.VMEM((128,), jnp.int32),
        pltpu.VMEM((128,), jnp.int32),
        pltpu.VMEM((128,), jnp.int32),
        pltpu.VMEM((128,), jnp.int32),
    ],
)
def _remap(row_hbm, col_hbm, rowp_hbm, coll_hbm, rbuf, cbuf, rpbuf, clbuf):
  c, s = _worker_ids()
  half = c * _E_HALF
  sub = jnp.where(c == 0, _NU, 0)

  def chunk(g, _):
    m = g * 16 + s

    @pl.when(m < _NCHUNK)
    def _():
      eoff = half + m * 128
      pltpu.sync_copy(row_hbm.at[pl.ds(eoff, 128)], rbuf)
      pltpu.sync_copy(col_hbm.at[pl.ds(eoff, 128)], cbuf)

      def grp(k, _):
        o = k * 16
        rv = rbuf[pl.ds(o, 16)]
        rpbuf[pl.ds(o, 16)] = jnp.where(rv >= _NU, rv + _ITEM_SHIFT, rv)
        clbuf[pl.ds(o, 16)] = cbuf[pl.ds(o, 16)] - jnp.full((16,), sub,
                                                            jnp.int32)
        return 0
      lax.fori_loop(0, 8, grp, 0)
      pltpu.sync_copy(rpbuf, rowp_hbm.at[pl.ds(eoff, 128)])
      pltpu.sync_copy(clbuf, coll_hbm.at[pl.ds(eoff, 128)])
    return 0
  lax.fori_loop(0, _GMAX, chunk, 0)


@functools.partial(
    pl.kernel,
    out_type=jax.ShapeDtypeStruct((_PAD_N, _D), jnp.float32),
    mesh=plsc.VectorSubcoreMesh(**_MESH),
    compiler_params=_SC_PARAMS,
    scratch_types=[
        pltpu.VMEM_SHARED((30208, _D), jnp.float32),
        pltpu.VMEM((128,), jnp.int32),
        pltpu.VMEM((128,), jnp.int32),
        pltpu.VMEM((128, _D), jnp.float32),
        pltpu.SemaphoreType.DMA,
    ],
)
def _propagate_sc(y_hbm, rowp_hbm, coll_hbm, a_hbm,
                  acc, ridx, lidx, rbuf, sem):
  c, s = _worker_ids()
  half = c * _E_HALF
  zeros16 = jnp.zeros((16,), jnp.float32)

  def zb(k, _):
    r = k // 4
    o = (k % 4) * 16
    rbuf[r, pl.ds(o, 16)] = zeros16
    return 0
  lax.fori_loop(0, 512, zb, 0)

  nzc = jnp.where(c == 0, 160, 236)

  def za(g, _):
    t = g * 16 + s

    @pl.when(t < nzc)
    def _():
      pltpu.sync_copy(rbuf, acc.at[pl.ds(t * 128, 128), :])
    return 0
  lax.fori_loop(0, 15, za, 0)
  plsc.subcore_barrier()

  def chunk(g, _):
    m = g * 16 + s

    @pl.when(m < _NCHUNK)
    def _():
      eoff = half + m * 128
      pltpu.sync_copy(rowp_hbm.at[pl.ds(eoff, 128)], ridx)
      pltpu.sync_copy(coll_hbm.at[pl.ds(eoff, 128)], lidx)
      pltpu.async_copy(y_hbm.at[ridx], rbuf, sem).wait()
      pltpu.sync_copy(rbuf, acc.at[lidx], add=True)
    return 0
  lax.fori_loop(0, _GMAX, chunk, 0)
  plsc.subcore_barrier()

  base = jnp.where(c == 0, _PAD_U, 0)

  def wb(g, _):
    t = g * 16 + s

    @pl.when(t < nzc)
    def _():
      pltpu.sync_copy(acc.at[pl.ds(t * 128, 128), :],
                      a_hbm.at[pl.ds(base + t * 128, 128), :])
    return 0
  lax.fori_loop(0, 15, wb, 0)


@functools.partial(
    pl.kernel,
    out_type=[
        jax.ShapeDtypeStruct((_B, _D), jnp.float32),
        jax.ShapeDtypeStruct((_B, _D), jnp.float32),
        jax.ShapeDtypeStruct((_B, _D), jnp.float32),
    ],
    mesh=plsc.VectorSubcoreMesh(**_MESH),
    compiler_params=_SC_PARAMS,
    scratch_types=[
        pltpu.VMEM((128,), jnp.int32),
        pltpu.VMEM((128,), jnp.int32),
        pltpu.VMEM((128,), jnp.int32),
        pltpu.VMEM((128, _D), jnp.float32),
        pltpu.VMEM((128, _D), jnp.float32),
        pltpu.VMEM((128, _D), jnp.float32),
        pltpu.SemaphoreType.DMA,
    ],
)
def _batch_gather(z_hbm, u_hbm, i_hbm, j_hbm, zu_hbm, zi_hbm, zj_hbm,
                  uix, iix, jix, zu, zi, zj, sem):
  c, s = _worker_ids()
  w = s * 2 + c
  shift = jnp.full((16,), _PAD_U, jnp.int32)

  def chunk(k, _):
    boff = w * 512 + k * 128
    pltpu.sync_copy(u_hbm.at[pl.ds(boff, 128)], uix)
    pltpu.sync_copy(i_hbm.at[pl.ds(boff, 128)], iix)
    pltpu.sync_copy(j_hbm.at[pl.ds(boff, 128)], jix)

    def off(kk, _):
      o = kk * 16
      iix[pl.ds(o, 16)] = iix[pl.ds(o, 16)] + shift
      jix[pl.ds(o, 16)] = jix[pl.ds(o, 16)] + shift
      return 0
    lax.fori_loop(0, 8, off, 0)

    pltpu.async_copy(z_hbm.at[uix], zu, sem).wait()
    pltpu.async_copy(z_hbm.at[iix], zi, sem).wait()
    pltpu.async_copy(z_hbm.at[jix], zj, sem).wait()

    pltpu.sync_copy(zu, zu_hbm.at[pl.ds(boff, 128), :])
    pltpu.sync_copy(zi, zi_hbm.at[pl.ds(boff, 128), :])
    pltpu.sync_copy(zj, zj_hbm.at[pl.ds(boff, 128), :])
    return 0
  lax.fori_loop(0, 4, chunk, 0)


_ROWS_BLK = 1024
_NBLK = _PAD_N // _ROWS_BLK


def _scale(x, deg2, power):
  def body(x_ref, d_ref, o_ref):
    d = d_ref[...]
    if power == -0.5:
      sc = lax.rsqrt(d)
    else:
      sc = 1.0 / d
    sc = jnp.where(d > 0, sc, 0.0)
    o_ref[...] = x_ref[...] * sc

  return pl.pallas_call(
      body,
      grid=(_NBLK,),
      in_specs=[
          pl.BlockSpec((_ROWS_BLK, _D), lambda g: (g, 0)),
          pl.BlockSpec((_ROWS_BLK, 1), lambda g: (g, 0)),
      ],
      out_specs=pl.BlockSpec((_ROWS_BLK, _D), lambda g: (g, 0)),
      out_shape=jax.ShapeDtypeStruct((_PAD_N, _D), jnp.float32),
  )(x, deg2)


def _fuse_dense(ego_p, ego_n, a1, a2, a3, deg2,
                w0t, b0, w1t, b1, wat, ba, wqt):
  def body(ep_ref, en_ref, a1_ref, a2_ref, a3_ref, d_ref,
           w0_ref, b0_ref, w1_ref, b1_ref, wa_ref, ba_ref, wq_ref, o_ref):
    d = d_ref[...]
    dinv = jnp.where(d > 0, lax.rsqrt(d), 0.0)
    zp = (ep_ref[...] + dinv * (a1_ref[...] + a2_ref[...] + a3_ref[...])) * 0.25
    h = jnp.maximum(
        jnp.dot(en_ref[...], w0_ref[...],
                preferred_element_type=jnp.float32) + b0_ref[...], 0.0)
    zn = jnp.maximum(
        jnp.dot(h, w1_ref[...],
                preferred_element_type=jnp.float32) + b1_ref[...], 0.0)
    hp = jnp.tanh(jnp.dot(zp, wa_ref[...],
                          preferred_element_type=jnp.float32) + ba_ref[...])
    hn = jnp.tanh(jnp.dot(zn, wa_ref[...],
                          preferred_element_type=jnp.float32) + ba_ref[...])
    wp = jnp.dot(hp, wq_ref[...], preferred_element_type=jnp.float32)
    wn = jnp.dot(hn, wq_ref[...], preferred_element_type=jnp.float32)
    mx = jnp.maximum(wp, wn)
    e_p = jnp.exp(wp - mx)
    e_n = jnp.exp(wn - mx)
    ap = e_p / (e_p + e_n)
    o_ref[...] = ap * zp + (1.0 - ap) * zn

  blk = lambda g: (g, 0)
  zero = lambda g: (0, 0)
  return pl.pallas_call(
      body,
      grid=(_NBLK,),
      in_specs=[
          pl.BlockSpec((_ROWS_BLK, _D), blk),
          pl.BlockSpec((_ROWS_BLK, _D), blk),
          pl.BlockSpec((_ROWS_BLK, _D), blk),
          pl.BlockSpec((_ROWS_BLK, _D), blk),
          pl.BlockSpec((_ROWS_BLK, _D), blk),
          pl.BlockSpec((_ROWS_BLK, 1), blk),
          pl.BlockSpec((_D, _D), zero),
          pl.BlockSpec((1, _D), zero),
          pl.BlockSpec((_D, _D), zero),
          pl.BlockSpec((1, _D), zero),
          pl.BlockSpec((_D, _D), zero),
          pl.BlockSpec((1, _D), zero),
          pl.BlockSpec((_D, 1), zero),
      ],
      out_specs=pl.BlockSpec((_ROWS_BLK, _D), blk),
      out_shape=jax.ShapeDtypeStruct((_PAD_N, _D), jnp.float32),
  )(ego_p, ego_n, a1, a2, a3, deg2, w0t, b0, w1t, b1, wat, ba, wqt)


def _loss(zu, zi, zj, sgn2):
  nblk = 16
  rows = _B // nblk

  def body(u_ref, i_ref, j_ref, s_ref, o_ref):
    g = pl.program_id(0)
    u = u_ref[...]
    i = i_ref[...]
    j = j_ref[...]
    pos = jnp.sum(u * i, axis=1, keepdims=True)
    neg = jnp.sum(u * j, axis=1, keepdims=True)
    regs = jnp.sum(u * u + i * i + j * j, axis=1, keepdims=True)
    sc = s_ref[...] * pos - neg
    sig = 1.0 / (1.0 + jnp.exp(-sc))
    l = jnp.log(_GAMMA + sig)
    part = (-jnp.sum(l) + _REG * jnp.sum(regs)) * (1.0 / _B)

    @pl.when(g == 0)
    def _():
      o_ref[...] = jnp.zeros((1, 1), jnp.float32)
    o_ref[...] = o_ref[...] + jnp.reshape(part, (1, 1))

  blk = lambda g: (g, 0)
  return pl.pallas_call(
      body,
      grid=(nblk,),
      in_specs=[
          pl.BlockSpec((rows, _D), blk),
          pl.BlockSpec((rows, _D), blk),
          pl.BlockSpec((rows, _D), blk),
          pl.BlockSpec((rows, 1), blk),
      ],
      out_specs=pl.BlockSpec((1, 1), lambda g: (0, 0)),
      out_shape=jax.ShapeDtypeStruct((1, 1), jnp.float32),
  )(zu, zi, zj, sgn2)


def kernel(u, i, j, sgn, edge_index, emb_pos_u, emb_pos_i, emb_neg_u,
           emb_neg_i, W_mlp0, b_mlp0, W_mlp1, b_mlp1, W_attn, b_attn, W_q):
  row = edge_index[0].astype(jnp.int32)
  col = edge_index[1].astype(jnp.int32)

  ego_p = jnp.zeros((_PAD_N, _D), jnp.float32)
  ego_p = ego_p.at[:_NU].set(emb_pos_u).at[_PAD_U:_PAD_U + _NV].set(emb_pos_i)
  ego_n = jnp.zeros((_PAD_N, _D), jnp.float32)
  ego_n = ego_n.at[:_NU].set(emb_neg_u).at[_PAD_U:_PAD_U + _NV].set(emb_neg_i)

  rowp, coll = _remap(row, col)
  deg64 = _propagate_sc(jnp.ones((_PAD_N, _D), jnp.float32), rowp, coll)
  deg2 = deg64[:, 0:1]

  y0 = _scale(ego_p, deg2, -0.5)
  a1 = _propagate_sc(y0, rowp, coll)
  y1 = _scale(a1, deg2, -1.0)
  a2 = _propagate_sc(y1, rowp, coll)
  y2 = _scale(a2, deg2, -1.0)
  a3 = _propagate_sc(y2, rowp, coll)

  z = _fuse_dense(ego_p, ego_n, a1, a2, a3, deg2,
                  W_mlp0.T, b_mlp0.reshape(1, _D),
                  W_mlp1.T, b_mlp1.reshape(1, _D),
                  W_attn.T, b_attn.reshape(1, _D),
                  W_q.reshape(1, _D).T)

  zu, zi, zj = _batch_gather(z, u.astype(jnp.int32),
                             i.astype(jnp.int32), j.astype(jnp.int32))

  out = _loss(zu, zi, zj, sgn.reshape(_B, 1))
  return out.reshape(())


# speedup vs baseline: 9.3431x; 1.0456x over previous
import functools

import jax
import jax.numpy as jnp
from jax import lax
from jax.experimental import pallas as pl
from jax.experimental.pallas import tpu as pltpu
from jax.experimental.pallas import tpu_sc as plsc

_NU = 30000
_NV = 20000
_N = 50000
_D = 64
_PAD_U = 30720
_PAD_N = 51200
_ITEM_SHIFT = _PAD_U - _NU
_E_HALF = 400000
_NCHUNK = _E_HALF // 128
_GMAX = (_NCHUNK + 15) // 16
_B = 16384
_REG = 0.05
_GAMMA = 1e-10

_EROWS = 12544
_EPAD = _EROWS * 64

_MESH = dict(core_axis_name="c", subcore_axis_name="s", num_cores=2,
             num_subcores=16)
_SC_PARAMS = pltpu.CompilerParams(use_tc_tiling_on_sc=False)


def _worker_ids():
  c = lax.axis_index("c")
  s = lax.axis_index("s")
  return c, s


@functools.partial(
    pl.kernel,
    out_type=[
        jax.ShapeDtypeStruct((_EPAD,), jnp.int32),
        jax.ShapeDtypeStruct((_EPAD,), jnp.int32),
    ],
    mesh=plsc.VectorSubcoreMesh(**_MESH),
    compiler_params=_SC_PARAMS,
    scratch_types=[
        pltpu.VMEM((128,), jnp.int32),
        pltpu.VMEM((128,), jnp.int32),
        pltpu.VMEM((128,), jnp.int32),
        pltpu.VMEM((128,), jnp.int32),
    ],
)
def _remap(row_hbm, col_hbm, rowp_hbm, coll_hbm, rbuf, cbuf, rpbuf, clbuf):
  c, s = _worker_ids()
  half = c * _E_HALF
  sub = jnp.where(c == 0, _NU, 0)

  def chunk(g, _):
    m = g * 16 + s

    @pl.when(m < _NCHUNK)
    def _():
      eoff = half + m * 128
      pltpu.sync_copy(row_hbm.at[pl.ds(eoff, 128)], rbuf)
      pltpu.sync_copy(col_hbm.at[pl.ds(eoff, 128)], cbuf)

      def grp(k, _):
        o = k * 16
        rv = rbuf[pl.ds(o, 16)]
        rpbuf[pl.ds(o, 16)] = jnp.where(rv >= _NU, rv + _ITEM_SHIFT, rv)
        clbuf[pl.ds(o, 16)] = cbuf[pl.ds(o, 16)] - jnp.full((16,), sub,
                                                            jnp.int32)
        return 0
      lax.fori_loop(0, 8, grp, 0)
      pltpu.sync_copy(rpbuf, rowp_hbm.at[pl.ds(eoff, 128)])
      pltpu.sync_copy(clbuf, coll_hbm.at[pl.ds(eoff, 128)])
    return 0
  lax.fori_loop(0, _GMAX, chunk, 0)


_CH = 64
_NCH64 = _E_HALF // _CH
_CPT = _NCH64 // 16
_CREM = _NCH64 - 16 * _CPT
_NBLK8 = (_CPT + 1 + 7) // 8
_ACC_ROWS = 30080


@functools.partial(
    pl.kernel,
    out_type=jax.ShapeDtypeStruct((_PAD_N, _D), jnp.float32),
    mesh=plsc.VectorSubcoreMesh(**_MESH),
    compiler_params=_SC_PARAMS,
    scratch_types=[
        pltpu.VMEM_SHARED((_ACC_ROWS, _D), jnp.float32),
        pltpu.VMEM((8, _CH), jnp.int32),
        pltpu.VMEM((8, _CH), jnp.int32),
        pltpu.VMEM((_CH,), jnp.int32),
        pltpu.VMEM((_CH,), jnp.int32),
        pltpu.VMEM((_CH,), jnp.int32),
        pltpu.VMEM((_CH,), jnp.int32),
        pltpu.VMEM((_CH, _D), jnp.float32),
        pltpu.VMEM((_CH, _D), jnp.float32),
        pltpu.SemaphoreType.DMA,
        pltpu.SemaphoreType.DMA,
    ],
)
def _propagate_sc(y_hbm, rowp_hbm, coll_hbm, a_hbm,
                  acc, ridxblk, collblk, ridxa, ridxb, lidxa, lidxb,
                  rbufa, rbufb, sema, semb):
  c, s = _worker_ids()
  zeros16 = jnp.zeros((16,), jnp.float32)

  def zb(k, _):
    rbufa[k // 4, pl.ds((k % 4) * 16, 16)] = zeros16
    return 0
  lax.fori_loop(0, _CH * 4, zb, 0)

  nzc = jnp.where(c == 0, 20480 // _CH, _ACC_ROWS // _CH)

  def za(g, _):
    t = g * 16 + s

    @pl.when(t < nzc)
    def _():
      pltpu.sync_copy(rbufa, acc.at[pl.ds(t * _CH, _CH), :])
    return 0
  lax.fori_loop(0, _ACC_ROWS // _CH // 16 + 1, za, 0)
  plsc.subcore_barrier()

  start = s * _CPT + jnp.minimum(s, _CREM)
  cnt = _CPT + jnp.where(s < _CREM, 1, 0)
  r0base = c * _NCH64 + start

  idxs = (ridxa, ridxb)
  lids = (lidxa, lidxb)
  bufs = (rbufa, rbufb)
  sems = (sema, semb)

  def block(b, _):
    base_n = b * 8

    @pl.when(base_n < cnt)
    def _():
      pltpu.sync_copy(rowp_hbm.at[pl.ds(r0base + base_n, 8), :], ridxblk)
      pltpu.sync_copy(coll_hbm.at[pl.ds(r0base + base_n, 8), :], collblk)

    for k in range(8):
      n = base_n + k
      p = (k + 1) % 2

      prev_n = n - 1

      @pl.when(jnp.logical_and(prev_n >= 0, prev_n < cnt))
      def _(p=p):
        pltpu.make_async_copy(y_hbm.at[idxs[p]], bufs[p], sems[p]).wait()
        pltpu.sync_copy(bufs[p], acc.at[lids[p]], add=True)

      @pl.when(n < cnt)
      def _(k=k):
        q = k % 2
        for o in range(4):
          idxs[q][pl.ds(o * 16, 16)] = ridxblk[k, pl.ds(o * 16, 16)]
          lids[q][pl.ds(o * 16, 16)] = collblk[k, pl.ds(o * 16, 16)]
        pltpu.async_copy(y_hbm.at[idxs[q]], bufs[q], sems[q])
    return 0
  lax.fori_loop(0, _NBLK8, block, 0)
  plsc.subcore_barrier()

  base = jnp.where(c == 0, _PAD_U, 0)

  def wb(g, _):
    t = g * 16 + s

    @pl.when(t < nzc)
    def _():
      pltpu.sync_copy(acc.at[pl.ds(t * _CH, _CH), :],
                      a_hbm.at[pl.ds(base + t * _CH, _CH), :])
    return 0
  lax.fori_loop(0, _ACC_ROWS // _CH // 16 + 1, wb, 0)


@functools.partial(
    pl.kernel,
    out_type=[
        jax.ShapeDtypeStruct((_B, _D), jnp.float32),
        jax.ShapeDtypeStruct((_B, _D), jnp.float32),
        jax.ShapeDtypeStruct((_B, _D), jnp.float32),
    ],
    mesh=plsc.VectorSubcoreMesh(**_MESH),
    compiler_params=_SC_PARAMS,
    scratch_types=[
        pltpu.VMEM((128,), jnp.int32),
        pltpu.VMEM((128,), jnp.int32),
        pltpu.VMEM((128,), jnp.int32),
        pltpu.VMEM((128, _D), jnp.float32),
        pltpu.VMEM((128, _D), jnp.float32),
        pltpu.VMEM((128, _D), jnp.float32),
        pltpu.SemaphoreType.DMA,
    ],
)
def _batch_gather(z_hbm, u_hbm, i_hbm, j_hbm, zu_hbm, zi_hbm, zj_hbm,
                  uix, iix, jix, zu, zi, zj, sem):
  c, s = _worker_ids()
  w = s * 2 + c
  shift = jnp.full((16,), _PAD_U, jnp.int32)

  def chunk(k, _):
    boff = w * 512 + k * 128
    pltpu.sync_copy(u_hbm.at[pl.ds(boff, 128)], uix)
    pltpu.sync_copy(i_hbm.at[pl.ds(boff, 128)], iix)
    pltpu.sync_copy(j_hbm.at[pl.ds(boff, 128)], jix)

    def off(kk, _):
      o = kk * 16
      iix[pl.ds(o, 16)] = iix[pl.ds(o, 16)] + shift
      jix[pl.ds(o, 16)] = jix[pl.ds(o, 16)] + shift
      return 0
    lax.fori_loop(0, 8, off, 0)

    pltpu.async_copy(z_hbm.at[uix], zu, sem).wait()
    pltpu.async_copy(z_hbm.at[iix], zi, sem).wait()
    pltpu.async_copy(z_hbm.at[jix], zj, sem).wait()

    pltpu.sync_copy(zu, zu_hbm.at[pl.ds(boff, 128), :])
    pltpu.sync_copy(zi, zi_hbm.at[pl.ds(boff, 128), :])
    pltpu.sync_copy(zj, zj_hbm.at[pl.ds(boff, 128), :])
    return 0
  lax.fori_loop(0, 4, chunk, 0)


_ROWS_BLK = 1024
_NBLK = _PAD_N // _ROWS_BLK


def _scale(x, deg2, power):
  def body(x_ref, d_ref, o_ref):
    d = d_ref[...]
    if power == -0.5:
      sc = lax.rsqrt(d)
    else:
      sc = 1.0 / d
    sc = jnp.where(d > 0, sc, 0.0)
    o_ref[...] = x_ref[...] * sc

  return pl.pallas_call(
      body,
      grid=(_NBLK,),
      in_specs=[
          pl.BlockSpec((_ROWS_BLK, _D), lambda g: (g, 0)),
          pl.BlockSpec((_ROWS_BLK, 1), lambda g: (g, 0)),
      ],
      out_specs=pl.BlockSpec((_ROWS_BLK, _D), lambda g: (g, 0)),
      out_shape=jax.ShapeDtypeStruct((_PAD_N, _D), jnp.float32),
  )(x, deg2)


def _fuse_dense(ego_p, ego_n, a1, a2, a3, deg2,
                w0t, b0, w1t, b1, wat, ba, wqt):
  def body(ep_ref, en_ref, a1_ref, a2_ref, a3_ref, d_ref,
           w0_ref, b0_ref, w1_ref, b1_ref, wa_ref, ba_ref, wq_ref, o_ref):
    d = d_ref[...]
    dinv = jnp.where(d > 0, lax.rsqrt(d), 0.0)
    zp = (ep_ref[...] + dinv * (a1_ref[...] + a2_ref[...] + a3_ref[...])) * 0.25
    h = jnp.maximum(
        jnp.dot(en_ref[...], w0_ref[...],
                preferred_element_type=jnp.float32) + b0_ref[...], 0.0)
    zn = jnp.maximum(
        jnp.dot(h, w1_ref[...],
                preferred_element_type=jnp.float32) + b1_ref[...], 0.0)
    hp = jnp.tanh(jnp.dot(zp, wa_ref[...],
                          preferred_element_type=jnp.float32) + ba_ref[...])
    hn = jnp.tanh(jnp.dot(zn, wa_ref[...],
                          preferred_element_type=jnp.float32) + ba_ref[...])
    wp = jnp.dot(hp, wq_ref[...], preferred_element_type=jnp.float32)
    wn = jnp.dot(hn, wq_ref[...], preferred_element_type=jnp.float32)
    mx = jnp.maximum(wp, wn)
    e_p = jnp.exp(wp - mx)
    e_n = jnp.exp(wn - mx)
    ap = e_p / (e_p + e_n)
    o_ref[...] = ap * zp + (1.0 - ap) * zn

  blk = lambda g: (g, 0)
  zero = lambda g: (0, 0)
  return pl.pallas_call(
      body,
      grid=(_NBLK,),
      in_specs=[
          pl.BlockSpec((_ROWS_BLK, _D), blk),
          pl.BlockSpec((_ROWS_BLK, _D), blk),
          pl.BlockSpec((_ROWS_BLK, _D), blk),
          pl.BlockSpec((_ROWS_BLK, _D), blk),
          pl.BlockSpec((_ROWS_BLK, _D), blk),
          pl.BlockSpec((_ROWS_BLK, 1), blk),
          pl.BlockSpec((_D, _D), zero),
          pl.BlockSpec((1, _D), zero),
          pl.BlockSpec((_D, _D), zero),
          pl.BlockSpec((1, _D), zero),
          pl.BlockSpec((_D, _D), zero),
          pl.BlockSpec((1, _D), zero),
          pl.BlockSpec((_D, 1), zero),
      ],
      out_specs=pl.BlockSpec((_ROWS_BLK, _D), blk),
      out_shape=jax.ShapeDtypeStruct((_PAD_N, _D), jnp.float32),
  )(ego_p, ego_n, a1, a2, a3, deg2, w0t, b0, w1t, b1, wat, ba, wqt)


def _loss(zu, zi, zj, sgn2):
  nblk = 16
  rows = _B // nblk

  def body(u_ref, i_ref, j_ref, s_ref, o_ref):
    g = pl.program_id(0)
    u = u_ref[...]
    i = i_ref[...]
    j = j_ref[...]
    pos = jnp.sum(u * i, axis=1, keepdims=True)
    neg = jnp.sum(u * j, axis=1, keepdims=True)
    regs = jnp.sum(u * u + i * i + j * j, axis=1, keepdims=True)
    sc = s_ref[...] * pos - neg
    sig = 1.0 / (1.0 + jnp.exp(-sc))
    l = jnp.log(_GAMMA + sig)
    part = (-jnp.sum(l) + _REG * jnp.sum(regs)) * (1.0 / _B)

    @pl.when(g == 0)
    def _():
      o_ref[...] = jnp.zeros((1, 1), jnp.float32)
    o_ref[...] = o_ref[...] + jnp.reshape(part, (1, 1))

  blk = lambda g: (g, 0)
  return pl.pallas_call(
      body,
      grid=(nblk,),
      in_specs=[
          pl.BlockSpec((rows, _D), blk),
          pl.BlockSpec((rows, _D), blk),
          pl.BlockSpec((rows, _D), blk),
          pl.BlockSpec((rows, 1), blk),
      ],
      out_specs=pl.BlockSpec((1, 1), lambda g: (0, 0)),
      out_shape=jax.ShapeDtypeStruct((1, 1), jnp.float32),
  )(zu, zi, zj, sgn2)


def kernel(u, i, j, sgn, edge_index, emb_pos_u, emb_pos_i, emb_neg_u,
           emb_neg_i, W_mlp0, b_mlp0, W_mlp1, b_mlp1, W_attn, b_attn, W_q):
  row = edge_index[0].astype(jnp.int32)
  col = edge_index[1].astype(jnp.int32)

  ego_p = jnp.zeros((_PAD_N, _D), jnp.float32)
  ego_p = ego_p.at[:_NU].set(emb_pos_u).at[_PAD_U:_PAD_U + _NV].set(emb_pos_i)
  ego_n = jnp.zeros((_PAD_N, _D), jnp.float32)
  ego_n = ego_n.at[:_NU].set(emb_neg_u).at[_PAD_U:_PAD_U + _NV].set(emb_neg_i)

  rowp, coll = _remap(row, col)
  rowp2 = rowp.reshape(_EROWS, 64)
  coll2 = coll.reshape(_EROWS, 64)
  deg64 = _propagate_sc(jnp.ones((_PAD_N, _D), jnp.float32), rowp2, coll2)
  deg2 = deg64[:, 0:1]

  y0 = _scale(ego_p, deg2, -0.5)
  a1 = _propagate_sc(y0, rowp2, coll2)
  y1 = _scale(a1, deg2, -1.0)
  a2 = _propagate_sc(y1, rowp2, coll2)
  y2 = _scale(a2, deg2, -1.0)
  a3 = _propagate_sc(y2, rowp2, coll2)

  z = _fuse_dense(ego_p, ego_n, a1, a2, a3, deg2,
                  W_mlp0.T, b_mlp0.reshape(1, _D),
                  W_mlp1.T, b_mlp1.reshape(1, _D),
                  W_attn.T, b_attn.reshape(1, _D),
                  W_q.reshape(1, _D).T)

  zu, zi, zj = _batch_gather(z, u.astype(jnp.int32),
                             i.astype(jnp.int32), j.astype(jnp.int32))

  out = _loss(zu, zi, zj, sgn.reshape(_B, 1))
  return out.reshape(())


# speedup vs baseline: 9.3434x; 1.0000x over previous
import functools

import jax
import jax.numpy as jnp
from jax import lax
from jax.experimental import pallas as pl
from jax.experimental.pallas import tpu as pltpu
from jax.experimental.pallas import tpu_sc as plsc

_NU = 30000
_NV = 20000
_N = 50000
_D = 64
_PAD_U = 30720
_PAD_N = 51200
_ITEM_SHIFT = _PAD_U - _NU
_E_HALF = 400000
_NCHUNK = _E_HALF // 128
_GMAX = (_NCHUNK + 15) // 16
_B = 16384
_REG = 0.05
_GAMMA = 1e-10

_EROWS = 12544
_EPAD = _EROWS * 64

_MESH = dict(core_axis_name="c", subcore_axis_name="s", num_cores=2,
             num_subcores=16)
_SC_PARAMS = pltpu.CompilerParams(use_tc_tiling_on_sc=False)


def _worker_ids():
  c = lax.axis_index("c")
  s = lax.axis_index("s")
  return c, s


@functools.partial(
    pl.kernel,
    out_type=[
        jax.ShapeDtypeStruct((_EPAD,), jnp.int32),
        jax.ShapeDtypeStruct((_EPAD,), jnp.int32),
    ],
    mesh=plsc.VectorSubcoreMesh(**_MESH),
    compiler_params=_SC_PARAMS,
    scratch_types=[
        pltpu.VMEM((128,), jnp.int32),
        pltpu.VMEM((128,), jnp.int32),
        pltpu.VMEM((128,), jnp.int32),
        pltpu.VMEM((128,), jnp.int32),
    ],
)
def _remap(row_hbm, col_hbm, rowp_hbm, coll_hbm, rbuf, cbuf, rpbuf, clbuf):
  c, s = _worker_ids()
  half = c * _E_HALF
  sub = jnp.where(c == 0, _NU, 0)

  def chunk(g, _):
    m = g * 16 + s

    @pl.when(m < _NCHUNK)
    def _():
      eoff = half + m * 128
      pltpu.sync_copy(row_hbm.at[pl.ds(eoff, 128)], rbuf)
      pltpu.sync_copy(col_hbm.at[pl.ds(eoff, 128)], cbuf)

      def grp(k, _):
        o = k * 16
        rv = rbuf[pl.ds(o, 16)]
        rpbuf[pl.ds(o, 16)] = jnp.where(rv >= _NU, rv + _ITEM_SHIFT, rv)
        clbuf[pl.ds(o, 16)] = cbuf[pl.ds(o, 16)] - jnp.full((16,), sub,
                                                            jnp.int32)
        return 0
      lax.fori_loop(0, 8, grp, 0)
      pltpu.sync_copy(rpbuf, rowp_hbm.at[pl.ds(eoff, 128)])
      pltpu.sync_copy(clbuf, coll_hbm.at[pl.ds(eoff, 128)])
    return 0
  lax.fori_loop(0, _GMAX, chunk, 0)


_CH = 64
_NCH64 = _E_HALF // _CH
_CPT = _NCH64 // 16
_CREM = _NCH64 - 16 * _CPT
_NBLK8 = (_CPT + 1 + 7) // 8
_ACC_ROWS = 30080


@functools.partial(
    pl.kernel,
    out_type=jax.ShapeDtypeStruct((_PAD_N, _D), jnp.float32),
    mesh=plsc.VectorSubcoreMesh(**_MESH),
    compiler_params=_SC_PARAMS,
    scratch_types=[
        pltpu.VMEM_SHARED((_ACC_ROWS, _D), jnp.float32),
        pltpu.VMEM((8, _CH), jnp.int32),
        pltpu.VMEM((8, _CH), jnp.int32),
        pltpu.VMEM((_CH,), jnp.int32),
        pltpu.VMEM((_CH,), jnp.int32),
        pltpu.VMEM((_CH,), jnp.int32),
        pltpu.VMEM((_CH,), jnp.int32),
        pltpu.VMEM((_CH, _D), jnp.float32),
        pltpu.VMEM((_CH, _D), jnp.float32),
        pltpu.SemaphoreType.DMA,
        pltpu.SemaphoreType.DMA,
        pltpu.SemaphoreType.DMA,
        pltpu.SemaphoreType.DMA,
    ],
)
def _propagate_sc(y_hbm, rowp_hbm, coll_hbm, a_hbm,
                  acc, ridxblk, collblk, ridxa, ridxb, lidxa, lidxb,
                  rbufa, rbufb, sema, semb, semsa, semsb):
  c, s = _worker_ids()
  zeros16 = jnp.zeros((16,), jnp.float32)

  def zb(k, _):
    rbufa[k // 4, pl.ds((k % 4) * 16, 16)] = zeros16
    return 0
  lax.fori_loop(0, _CH * 4, zb, 0)

  nzc = jnp.where(c == 0, 20480 // _CH, _ACC_ROWS // _CH)

  def za(g, _):
    t = g * 16 + s

    @pl.when(t < nzc)
    def _():
      pltpu.sync_copy(rbufa, acc.at[pl.ds(t * _CH, _CH), :])
    return 0
  lax.fori_loop(0, _ACC_ROWS // _CH // 16 + 1, za, 0)
  plsc.subcore_barrier()

  start = s * _CPT + jnp.minimum(s, _CREM)
  cnt = _CPT + jnp.where(s < _CREM, 1, 0)
  r0base = c * _NCH64 + start

  idxs = (ridxa, ridxb)
  lids = (lidxa, lidxb)
  bufs = (rbufa, rbufb)
  sems = (sema, semb)

  def block(b, _):
    base_n = b * 8

    @pl.when(base_n < cnt)
    def _():
      pltpu.sync_copy(rowp_hbm.at[pl.ds(r0base + base_n, 8), :], ridxblk)
      pltpu.sync_copy(coll_hbm.at[pl.ds(r0base + base_n, 8), :], collblk)

    for k in range(8):
      n = base_n + k
      p = (k + 1) % 2

      prev_n = n - 1

      @pl.when(jnp.logical_and(prev_n >= 0, prev_n < cnt))
      def _(p=p):
        pltpu.make_async_copy(y_hbm.at[idxs[p]], bufs[p], sems[p]).wait()
        pltpu.sync_copy(bufs[p], acc.at[lids[p]], add=True)

      @pl.when(n < cnt)
      def _(k=k):
        q = k % 2
        for o in range(4):
          idxs[q][pl.ds(o * 16, 16)] = ridxblk[k, pl.ds(o * 16, 16)]
          lids[q][pl.ds(o * 16, 16)] = collblk[k, pl.ds(o * 16, 16)]
        pltpu.async_copy(y_hbm.at[idxs[q]], bufs[q], sems[q])
    return 0
  lax.fori_loop(0, _NBLK8, block, 0)
  plsc.subcore_barrier()

  base = jnp.where(c == 0, _PAD_U, 0)

  def wb(g, _):
    t = g * 16 + s

    @pl.when(t < nzc)
    def _():
      pltpu.sync_copy(acc.at[pl.ds(t * _CH, _CH), :],
                      a_hbm.at[pl.ds(base + t * _CH, _CH), :])
    return 0
  lax.fori_loop(0, _ACC_ROWS // _CH // 16 + 1, wb, 0)


@functools.partial(
    pl.kernel,
    out_type=[
        jax.ShapeDtypeStruct((_B, _D), jnp.float32),
        jax.ShapeDtypeStruct((_B, _D), jnp.float32),
        jax.ShapeDtypeStruct((_B, _D), jnp.float32),
    ],
    mesh=plsc.VectorSubcoreMesh(**_MESH),
    compiler_params=_SC_PARAMS,
    scratch_types=[
        pltpu.VMEM((128,), jnp.int32),
        pltpu.VMEM((128,), jnp.int32),
        pltpu.VMEM((128,), jnp.int32),
        pltpu.VMEM((128, _D), jnp.float32),
        pltpu.VMEM((128, _D), jnp.float32),
        pltpu.VMEM((128, _D), jnp.float32),
        pltpu.SemaphoreType.DMA,
    ],
)
def _batch_gather(z_hbm, u_hbm, i_hbm, j_hbm, zu_hbm, zi_hbm, zj_hbm,
                  uix, iix, jix, zu, zi, zj, sem):
  c, s = _worker_ids()
  w = s * 2 + c
  shift = jnp.full((16,), _PAD_U, jnp.int32)

  def chunk(k, _):
    boff = w * 512 + k * 128
    pltpu.sync_copy(u_hbm.at[pl.ds(boff, 128)], uix)
    pltpu.sync_copy(i_hbm.at[pl.ds(boff, 128)], iix)
    pltpu.sync_copy(j_hbm.at[pl.ds(boff, 128)], jix)

    def off(kk, _):
      o = kk * 16
      iix[pl.ds(o, 16)] = iix[pl.ds(o, 16)] + shift
      jix[pl.ds(o, 16)] = jix[pl.ds(o, 16)] + shift
      return 0
    lax.fori_loop(0, 8, off, 0)

    pltpu.async_copy(z_hbm.at[uix], zu, sem).wait()
    pltpu.async_copy(z_hbm.at[iix], zi, sem).wait()
    pltpu.async_copy(z_hbm.at[jix], zj, sem).wait()

    pltpu.sync_copy(zu, zu_hbm.at[pl.ds(boff, 128), :])
    pltpu.sync_copy(zi, zi_hbm.at[pl.ds(boff, 128), :])
    pltpu.sync_copy(zj, zj_hbm.at[pl.ds(boff, 128), :])
    return 0
  lax.fori_loop(0, 4, chunk, 0)


_ROWS_BLK = 1024
_NBLK = _PAD_N // _ROWS_BLK


def _scale(x, deg2, power):
  def body(x_ref, d_ref, o_ref):
    d = d_ref[...]
    if power == -0.5:
      sc = lax.rsqrt(d)
    else:
      sc = 1.0 / d
    sc = jnp.where(d > 0, sc, 0.0)
    o_ref[...] = x_ref[...] * sc

  return pl.pallas_call(
      body,
      grid=(_NBLK,),
      in_specs=[
          pl.BlockSpec((_ROWS_BLK, _D), lambda g: (g, 0)),
          pl.BlockSpec((_ROWS_BLK, 1), lambda g: (g, 0)),
      ],
      out_specs=pl.BlockSpec((_ROWS_BLK, _D), lambda g: (g, 0)),
      out_shape=jax.ShapeDtypeStruct((_PAD_N, _D), jnp.float32),
  )(x, deg2)


def _fuse_dense(ego_p, ego_n, a1, a2, a3, deg2,
                w0t, b0, w1t, b1, wat, ba, wqt):
  def body(ep_ref, en_ref, a1_ref, a2_ref, a3_ref, d_ref,
           w0_ref, b0_ref, w1_ref, b1_ref, wa_ref, ba_ref, wq_ref, o_ref):
    d = d_ref[...]
    dinv = jnp.where(d > 0, lax.rsqrt(d), 0.0)
    zp = (ep_ref[...] + dinv * (a1_ref[...] + a2_ref[...] + a3_ref[...])) * 0.25
    h = jnp.maximum(
        jnp.dot(en_ref[...], w0_ref[...],
                preferred_element_type=jnp.float32) + b0_ref[...], 0.0)
    zn = jnp.maximum(
        jnp.dot(h, w1_ref[...],
                preferred_element_type=jnp.float32) + b1_ref[...], 0.0)
    hp = jnp.tanh(jnp.dot(zp, wa_ref[...],
                          preferred_element_type=jnp.float32) + ba_ref[...])
    hn = jnp.tanh(jnp.dot(zn, wa_ref[...],
                          preferred_element_type=jnp.float32) + ba_ref[...])
    wp = jnp.dot(hp, wq_ref[...], preferred_element_type=jnp.float32)
    wn = jnp.dot(hn, wq_ref[...], preferred_element_type=jnp.float32)
    mx = jnp.maximum(wp, wn)
    e_p = jnp.exp(wp - mx)
    e_n = jnp.exp(wn - mx)
    ap = e_p / (e_p + e_n)
    o_ref[...] = ap * zp + (1.0 - ap) * zn

  blk = lambda g: (g, 0)
  zero = lambda g: (0, 0)
  return pl.pallas_call(
      body,
      grid=(_NBLK,),
      in_specs=[
          pl.BlockSpec((_ROWS_BLK, _D), blk),
          pl.BlockSpec((_ROWS_BLK, _D), blk),
          pl.BlockSpec((_ROWS_BLK, _D), blk),
          pl.BlockSpec((_ROWS_BLK, _D), blk),
          pl.BlockSpec((_ROWS_BLK, _D), blk),
          pl.BlockSpec((_ROWS_BLK, 1), blk),
          pl.BlockSpec((_D, _D), zero),
          pl.BlockSpec((1, _D), zero),
          pl.BlockSpec((_D, _D), zero),
          pl.BlockSpec((1, _D), zero),
          pl.BlockSpec((_D, _D), zero),
          pl.BlockSpec((1, _D), zero),
          pl.BlockSpec((_D, 1), zero),
      ],
      out_specs=pl.BlockSpec((_ROWS_BLK, _D), blk),
      out_shape=jax.ShapeDtypeStruct((_PAD_N, _D), jnp.float32),
  )(ego_p, ego_n, a1, a2, a3, deg2, w0t, b0, w1t, b1, wat, ba, wqt)


def _loss(zu, zi, zj, sgn2):
  nblk = 16
  rows = _B // nblk

  def body(u_ref, i_ref, j_ref, s_ref, o_ref):
    g = pl.program_id(0)
    u = u_ref[...]
    i = i_ref[...]
    j = j_ref[...]
    pos = jnp.sum(u * i, axis=1, keepdims=True)
    neg = jnp.sum(u * j, axis=1, keepdims=True)
    regs = jnp.sum(u * u + i * i + j * j, axis=1, keepdims=True)
    sc = s_ref[...] * pos - neg
    sig = 1.0 / (1.0 + jnp.exp(-sc))
    l = jnp.log(_GAMMA + sig)
    part = (-jnp.sum(l) + _REG * jnp.sum(regs)) * (1.0 / _B)

    @pl.when(g == 0)
    def _():
      o_ref[...] = jnp.zeros((1, 1), jnp.float32)
    o_ref[...] = o_ref[...] + jnp.reshape(part, (1, 1))

  blk = lambda g: (g, 0)
  return pl.pallas_call(
      body,
      grid=(nblk,),
      in_specs=[
          pl.BlockSpec((rows, _D), blk),
          pl.BlockSpec((rows, _D), blk),
          pl.BlockSpec((rows, _D), blk),
          pl.BlockSpec((rows, 1), blk),
      ],
      out_specs=pl.BlockSpec((1, 1), lambda g: (0, 0)),
      out_shape=jax.ShapeDtypeStruct((1, 1), jnp.float32),
  )(zu, zi, zj, sgn2)


def kernel(u, i, j, sgn, edge_index, emb_pos_u, emb_pos_i, emb_neg_u,
           emb_neg_i, W_mlp0, b_mlp0, W_mlp1, b_mlp1, W_attn, b_attn, W_q):
  row = edge_index[0].astype(jnp.int32)
  col = edge_index[1].astype(jnp.int32)

  ego_p = jnp.zeros((_PAD_N, _D), jnp.float32)
  ego_p = ego_p.at[:_NU].set(emb_pos_u).at[_PAD_U:_PAD_U + _NV].set(emb_pos_i)
  ego_n = jnp.zeros((_PAD_N, _D), jnp.float32)
  ego_n = ego_n.at[:_NU].set(emb_neg_u).at[_PAD_U:_PAD_U + _NV].set(emb_neg_i)

  rowp, coll = _remap(row, col)
  rowp2 = rowp.reshape(_EROWS, 64)
  coll2 = coll.reshape(_EROWS, 64)
  deg64 = _propagate_sc(jnp.ones((_PAD_N, _D), jnp.float32), rowp2, coll2)
  deg2 = deg64[:, 0:1]

  y0 = _scale(ego_p, deg2, -0.5)
  a1 = _propagate_sc(y0, rowp2, coll2)
  y1 = _scale(a1, deg2, -1.0)
  a2 = _propagate_sc(y1, rowp2, coll2)
  y2 = _scale(a2, deg2, -1.0)
  a3 = _propagate_sc(y2, rowp2, coll2)

  z = _fuse_dense(ego_p, ego_n, a1, a2, a3, deg2,
                  W_mlp0.T, b_mlp0.reshape(1, _D),
                  W_mlp1.T, b_mlp1.reshape(1, _D),
                  W_attn.T, b_attn.reshape(1, _D),
                  W_q.reshape(1, _D).T)

  zu, zi, zj = _batch_gather(z, u.astype(jnp.int32),
                             i.astype(jnp.int32), j.astype(jnp.int32))

  out = _loss(zu, zi, zj, sgn.reshape(_B, 1))
  return out.reshape(())


# speedup vs baseline: 10.7156x; 1.1469x over previous
import functools

import jax
import jax.numpy as jnp
from jax import lax
from jax.experimental import pallas as pl
from jax.experimental.pallas import tpu as pltpu
from jax.experimental.pallas import tpu_sc as plsc

_NU = 30000
_NV = 20000
_N = 50000
_D = 64
_PAD_U = 30720
_PAD_N = 51200
_ITEM_SHIFT = _PAD_U - _NU
_E_HALF = 400000
_NCHUNK = _E_HALF // 128
_GMAX = (_NCHUNK + 15) // 16
_B = 16384
_REG = 0.05
_GAMMA = 1e-10

_EROWS = 12544
_EPAD = _EROWS * 64

_MESH = dict(core_axis_name="c", subcore_axis_name="s", num_cores=2,
             num_subcores=16)
_SC_PARAMS = pltpu.CompilerParams(use_tc_tiling_on_sc=False)


def _worker_ids():
  c = lax.axis_index("c")
  s = lax.axis_index("s")
  return c, s


@functools.partial(
    pl.kernel,
    out_type=[
        jax.ShapeDtypeStruct((_EPAD,), jnp.int32),
        jax.ShapeDtypeStruct((_EPAD,), jnp.int32),
    ],
    mesh=plsc.VectorSubcoreMesh(**_MESH),
    compiler_params=_SC_PARAMS,
    scratch_types=[
        pltpu.VMEM((128,), jnp.int32),
        pltpu.VMEM((128,), jnp.int32),
        pltpu.VMEM((128,), jnp.int32),
        pltpu.VMEM((128,), jnp.int32),
    ],
)
def _remap(row_hbm, col_hbm, rowp_hbm, coll_hbm, rbuf, cbuf, rpbuf, clbuf):
  c, s = _worker_ids()
  half = c * _E_HALF
  sub = jnp.where(c == 0, _NU, 0)

  def chunk(g, _):
    m = g * 16 + s

    @pl.when(m < _NCHUNK)
    def _():
      eoff = half + m * 128
      pltpu.sync_copy(row_hbm.at[pl.ds(eoff, 128)], rbuf)
      pltpu.sync_copy(col_hbm.at[pl.ds(eoff, 128)], cbuf)

      def grp(k, _):
        o = k * 16
        rv = rbuf[pl.ds(o, 16)]
        rpbuf[pl.ds(o, 16)] = jnp.where(rv >= _NU, rv + _ITEM_SHIFT, rv)
        clbuf[pl.ds(o, 16)] = cbuf[pl.ds(o, 16)] - jnp.full((16,), sub,
                                                            jnp.int32)
        return 0
      lax.fori_loop(0, 8, grp, 0)
      pltpu.sync_copy(rpbuf, rowp_hbm.at[pl.ds(eoff, 128)])
      pltpu.sync_copy(clbuf, coll_hbm.at[pl.ds(eoff, 128)])
    return 0
  lax.fori_loop(0, _GMAX, chunk, 0)


_CH = 64
_NCH64 = _E_HALF // _CH
_CPT = _NCH64 // 16
_CREM = _NCH64 - 16 * _CPT
_NBLK8 = (_CPT + 1 + 7) // 8
_ACC_ROWS = 30080


@functools.partial(
    pl.kernel,
    out_type=jax.ShapeDtypeStruct((_PAD_N, _D), jnp.float32),
    mesh=plsc.VectorSubcoreMesh(**_MESH),
    compiler_params=_SC_PARAMS,
    scratch_types=[
        pltpu.VMEM_SHARED((_ACC_ROWS, _D), jnp.float32),
        pltpu.VMEM((8, _CH), jnp.int32),
        pltpu.VMEM((8, _CH), jnp.int32),
        pltpu.VMEM((_CH,), jnp.int32),
        pltpu.VMEM((_CH,), jnp.int32),
        pltpu.VMEM((_CH,), jnp.int32),
        pltpu.VMEM((_CH,), jnp.int32),
        pltpu.VMEM((_CH, _D), jnp.float32),
        pltpu.VMEM((_CH, _D), jnp.float32),
        pltpu.SemaphoreType.DMA,
        pltpu.SemaphoreType.DMA,
        pltpu.SemaphoreType.DMA,
        pltpu.SemaphoreType.DMA,
    ],
)
def _propagate_sc(y_hbm, rowp_hbm, coll_hbm, a_hbm,
                  acc, ridxblk, collblk, ridxa, ridxb, lidxa, lidxb,
                  rbufa, rbufb, sema, semb, semsa, semsb):
  c, s = _worker_ids()
  zeros16 = jnp.zeros((16,), jnp.float32)

  def zb(k, _):
    rbufa[k // 4, pl.ds((k % 4) * 16, 16)] = zeros16
    return 0
  lax.fori_loop(0, _CH * 4, zb, 0)

  nzc = jnp.where(c == 0, 20480 // _CH, _ACC_ROWS // _CH)

  def za(g, _):
    t = g * 16 + s

    @pl.when(t < nzc)
    def _():
      pltpu.sync_copy(rbufa, acc.at[pl.ds(t * _CH, _CH), :])
    return 0
  lax.fori_loop(0, _ACC_ROWS // _CH // 16 + 1, za, 0)
  plsc.subcore_barrier()

  start = s * _CPT + jnp.minimum(s, _CREM)
  cnt = _CPT + jnp.where(s < _CREM, 1, 0)
  r0base = c * _NCH64 + start

  idxs = (ridxa, ridxb)
  lids = (lidxa, lidxb)
  bufs = (rbufa, rbufb)
  sems = (sema, semb)
  ssems = (semsa, semsb)

  def block(b, _):
    base_n = b * 8

    @pl.when(base_n < cnt)
    def _():
      pltpu.sync_copy(rowp_hbm.at[pl.ds(r0base + base_n, 8), :], ridxblk)
      pltpu.sync_copy(coll_hbm.at[pl.ds(r0base + base_n, 8), :], collblk)

    for k in range(8):
      n = base_n + k
      p = (k + 1) % 2

      prev_n = n - 1

      @pl.when(jnp.logical_and(prev_n >= 0, prev_n < cnt))
      def _(p=p):
        pltpu.make_async_copy(y_hbm.at[idxs[p]], bufs[p], sems[p]).wait()
        pltpu.async_copy(bufs[p], acc.at[lids[p]], ssems[p], add=True)

      @pl.when(n < cnt)
      def _(k=k):
        q = k % 2

        @pl.when(n - 2 >= 0)
        def _():
          pltpu.make_async_copy(bufs[q], acc.at[lids[q]], ssems[q]).wait()
        for o in range(4):
          idxs[q][pl.ds(o * 16, 16)] = ridxblk[k, pl.ds(o * 16, 16)]
          lids[q][pl.ds(o * 16, 16)] = collblk[k, pl.ds(o * 16, 16)]
        pltpu.async_copy(y_hbm.at[idxs[q]], bufs[q], sems[q])
    return 0
  lax.fori_loop(0, _NBLK8, block, 0)

  @pl.when(cnt >= 2)
  def _():
    pltpu.make_async_copy(bufs[0], acc.at[lids[0]], ssems[0]).wait()

  @pl.when(cnt >= 1)
  def _():
    pltpu.make_async_copy(bufs[1], acc.at[lids[1]], ssems[1]).wait()
  plsc.subcore_barrier()

  base = jnp.where(c == 0, _PAD_U, 0)

  def wb(g, _):
    t = g * 16 + s

    @pl.when(t < nzc)
    def _():
      pltpu.sync_copy(acc.at[pl.ds(t * _CH, _CH), :],
                      a_hbm.at[pl.ds(base + t * _CH, _CH), :])
    return 0
  lax.fori_loop(0, _ACC_ROWS // _CH // 16 + 1, wb, 0)


@functools.partial(
    pl.kernel,
    out_type=[
        jax.ShapeDtypeStruct((_B, _D), jnp.float32),
        jax.ShapeDtypeStruct((_B, _D), jnp.float32),
        jax.ShapeDtypeStruct((_B, _D), jnp.float32),
    ],
    mesh=plsc.VectorSubcoreMesh(**_MESH),
    compiler_params=_SC_PARAMS,
    scratch_types=[
        pltpu.VMEM((128,), jnp.int32),
        pltpu.VMEM((128,), jnp.int32),
        pltpu.VMEM((128,), jnp.int32),
        pltpu.VMEM((128, _D), jnp.float32),
        pltpu.VMEM((128, _D), jnp.float32),
        pltpu.VMEM((128, _D), jnp.float32),
        pltpu.SemaphoreType.DMA,
    ],
)
def _batch_gather(z_hbm, u_hbm, i_hbm, j_hbm, zu_hbm, zi_hbm, zj_hbm,
                  uix, iix, jix, zu, zi, zj, sem):
  c, s = _worker_ids()
  w = s * 2 + c
  shift = jnp.full((16,), _PAD_U, jnp.int32)

  def chunk(k, _):
    boff = w * 512 + k * 128
    pltpu.sync_copy(u_hbm.at[pl.ds(boff, 128)], uix)
    pltpu.sync_copy(i_hbm.at[pl.ds(boff, 128)], iix)
    pltpu.sync_copy(j_hbm.at[pl.ds(boff, 128)], jix)

    def off(kk, _):
      o = kk * 16
      iix[pl.ds(o, 16)] = iix[pl.ds(o, 16)] + shift
      jix[pl.ds(o, 16)] = jix[pl.ds(o, 16)] + shift
      return 0
    lax.fori_loop(0, 8, off, 0)

    pltpu.async_copy(z_hbm.at[uix], zu, sem).wait()
    pltpu.async_copy(z_hbm.at[iix], zi, sem).wait()
    pltpu.async_copy(z_hbm.at[jix], zj, sem).wait()

    pltpu.sync_copy(zu, zu_hbm.at[pl.ds(boff, 128), :])
    pltpu.sync_copy(zi, zi_hbm.at[pl.ds(boff, 128), :])
    pltpu.sync_copy(zj, zj_hbm.at[pl.ds(boff, 128), :])
    return 0
  lax.fori_loop(0, 4, chunk, 0)


_ROWS_BLK = 1024
_NBLK = _PAD_N // _ROWS_BLK


def _scale(x, deg2, power):
  def body(x_ref, d_ref, o_ref):
    d = d_ref[...]
    if power == -0.5:
      sc = lax.rsqrt(d)
    else:
      sc = 1.0 / d
    sc = jnp.where(d > 0, sc, 0.0)
    o_ref[...] = x_ref[...] * sc

  return pl.pallas_call(
      body,
      grid=(_NBLK,),
      in_specs=[
          pl.BlockSpec((_ROWS_BLK, _D), lambda g: (g, 0)),
          pl.BlockSpec((_ROWS_BLK, 1), lambda g: (g, 0)),
      ],
      out_specs=pl.BlockSpec((_ROWS_BLK, _D), lambda g: (g, 0)),
      out_shape=jax.ShapeDtypeStruct((_PAD_N, _D), jnp.float32),
  )(x, deg2)


def _fuse_dense(ego_p, ego_n, a1, a2, a3, deg2,
                w0t, b0, w1t, b1, wat, ba, wqt):
  def body(ep_ref, en_ref, a1_ref, a2_ref, a3_ref, d_ref,
           w0_ref, b0_ref, w1_ref, b1_ref, wa_ref, ba_ref, wq_ref, o_ref):
    d = d_ref[...]
    dinv = jnp.where(d > 0, lax.rsqrt(d), 0.0)
    zp = (ep_ref[...] + dinv * (a1_ref[...] + a2_ref[...] + a3_ref[...])) * 0.25
    h = jnp.maximum(
        jnp.dot(en_ref[...], w0_ref[...],
                preferred_element_type=jnp.float32) + b0_ref[...], 0.0)
    zn = jnp.maximum(
        jnp.dot(h, w1_ref[...],
                preferred_element_type=jnp.float32) + b1_ref[...], 0.0)
    hp = jnp.tanh(jnp.dot(zp, wa_ref[...],
                          preferred_element_type=jnp.float32) + ba_ref[...])
    hn = jnp.tanh(jnp.dot(zn, wa_ref[...],
                          preferred_element_type=jnp.float32) + ba_ref[...])
    wp = jnp.dot(hp, wq_ref[...], preferred_element_type=jnp.float32)
    wn = jnp.dot(hn, wq_ref[...], preferred_element_type=jnp.float32)
    mx = jnp.maximum(wp, wn)
    e_p = jnp.exp(wp - mx)
    e_n = jnp.exp(wn - mx)
    ap = e_p / (e_p + e_n)
    o_ref[...] = ap * zp + (1.0 - ap) * zn

  blk = lambda g: (g, 0)
  zero = lambda g: (0, 0)
  return pl.pallas_call(
      body,
      grid=(_NBLK,),
      in_specs=[
          pl.BlockSpec((_ROWS_BLK, _D), blk),
          pl.BlockSpec((_ROWS_BLK, _D), blk),
          pl.BlockSpec((_ROWS_BLK, _D), blk),
          pl.BlockSpec((_ROWS_BLK, _D), blk),
          pl.BlockSpec((_ROWS_BLK, _D), blk),
          pl.BlockSpec((_ROWS_BLK, 1), blk),
          pl.BlockSpec((_D, _D), zero),
          pl.BlockSpec((1, _D), zero),
          pl.BlockSpec((_D, _D), zero),
          pl.BlockSpec((1, _D), zero),
          pl.BlockSpec((_D, _D), zero),
          pl.BlockSpec((1, _D), zero),
          pl.BlockSpec((_D, 1), zero),
      ],
      out_specs=pl.BlockSpec((_ROWS_BLK, _D), blk),
      out_shape=jax.ShapeDtypeStruct((_PAD_N, _D), jnp.float32),
  )(ego_p, ego_n, a1, a2, a3, deg2, w0t, b0, w1t, b1, wat, ba, wqt)


def _loss(zu, zi, zj, sgn2):
  nblk = 16
  rows = _B // nblk

  def body(u_ref, i_ref, j_ref, s_ref, o_ref):
    g = pl.program_id(0)
    u = u_ref[...]
    i = i_ref[...]
    j = j_ref[...]
    pos = jnp.sum(u * i, axis=1, keepdims=True)
    neg = jnp.sum(u * j, axis=1, keepdims=True)
    regs = jnp.sum(u * u + i * i + j * j, axis=1, keepdims=True)
    sc = s_ref[...] * pos - neg
    sig = 1.0 / (1.0 + jnp.exp(-sc))
    l = jnp.log(_GAMMA + sig)
    part = (-jnp.sum(l) + _REG * jnp.sum(regs)) * (1.0 / _B)

    @pl.when(g == 0)
    def _():
      o_ref[...] = jnp.zeros((1, 1), jnp.float32)
    o_ref[...] = o_ref[...] + jnp.reshape(part, (1, 1))

  blk = lambda g: (g, 0)
  return pl.pallas_call(
      body,
      grid=(nblk,),
      in_specs=[
          pl.BlockSpec((rows, _D), blk),
          pl.BlockSpec((rows, _D), blk),
          pl.BlockSpec((rows, _D), blk),
          pl.BlockSpec((rows, 1), blk),
      ],
      out_specs=pl.BlockSpec((1, 1), lambda g: (0, 0)),
      out_shape=jax.ShapeDtypeStruct((1, 1), jnp.float32),
  )(zu, zi, zj, sgn2)


def kernel(u, i, j, sgn, edge_index, emb_pos_u, emb_pos_i, emb_neg_u,
           emb_neg_i, W_mlp0, b_mlp0, W_mlp1, b_mlp1, W_attn, b_attn, W_q):
  row = edge_index[0].astype(jnp.int32)
  col = edge_index[1].astype(jnp.int32)

  ego_p = jnp.zeros((_PAD_N, _D), jnp.float32)
  ego_p = ego_p.at[:_NU].set(emb_pos_u).at[_PAD_U:_PAD_U + _NV].set(emb_pos_i)
  ego_n = jnp.zeros((_PAD_N, _D), jnp.float32)
  ego_n = ego_n.at[:_NU].set(emb_neg_u).at[_PAD_U:_PAD_U + _NV].set(emb_neg_i)

  rowp, coll = _remap(row, col)
  rowp2 = rowp.reshape(_EROWS, 64)
  coll2 = coll.reshape(_EROWS, 64)
  deg64 = _propagate_sc(jnp.ones((_PAD_N, _D), jnp.float32), rowp2, coll2)
  deg2 = deg64[:, 0:1]

  y0 = _scale(ego_p, deg2, -0.5)
  a1 = _propagate_sc(y0, rowp2, coll2)
  y1 = _scale(a1, deg2, -1.0)
  a2 = _propagate_sc(y1, rowp2, coll2)
  y2 = _scale(a2, deg2, -1.0)
  a3 = _propagate_sc(y2, rowp2, coll2)

  z = _fuse_dense(ego_p, ego_n, a1, a2, a3, deg2,
                  W_mlp0.T, b_mlp0.reshape(1, _D),
                  W_mlp1.T, b_mlp1.reshape(1, _D),
                  W_attn.T, b_attn.reshape(1, _D),
                  W_q.reshape(1, _D).T)

  zu, zi, zj = _batch_gather(z, u.astype(jnp.int32),
                             i.astype(jnp.int32), j.astype(jnp.int32))

  out = _loss(zu, zi, zj, sgn.reshape(_B, 1))
  return out.reshape(())


# speedup vs baseline: 12.3566x; 1.1531x over previous
import functools

import jax
import jax.numpy as jnp
from jax import lax
from jax.experimental import pallas as pl
from jax.experimental.pallas import tpu as pltpu
from jax.experimental.pallas import tpu_sc as plsc

_NU = 30000
_NV = 20000
_N = 50000
_D = 64
_PAD_U = 30720
_PAD_N = 51200
_ITEM_SHIFT = _PAD_U - _NU
_E_HALF = 400000
_NCHUNK = _E_HALF // 128
_GMAX = (_NCHUNK + 15) // 16
_B = 16384
_REG = 0.05
_GAMMA = 1e-10

_EROWS = 12544
_EPAD = _EROWS * 64

_MESH = dict(core_axis_name="c", subcore_axis_name="s", num_cores=2,
             num_subcores=16)
_SC_PARAMS = pltpu.CompilerParams(use_tc_tiling_on_sc=False)


def _worker_ids():
  c = lax.axis_index("c")
  s = lax.axis_index("s")
  return c, s


@functools.partial(
    pl.kernel,
    out_type=[
        jax.ShapeDtypeStruct((_EPAD,), jnp.int32),
        jax.ShapeDtypeStruct((_EPAD,), jnp.int32),
    ],
    mesh=plsc.VectorSubcoreMesh(**_MESH),
    compiler_params=_SC_PARAMS,
    scratch_types=[
        pltpu.VMEM((128,), jnp.int32),
        pltpu.VMEM((128,), jnp.int32),
        pltpu.VMEM((128,), jnp.int32),
        pltpu.VMEM((128,), jnp.int32),
    ],
)
def _remap(row_hbm, col_hbm, rowp_hbm, coll_hbm, rbuf, cbuf, rpbuf, clbuf):
  c, s = _worker_ids()
  half = c * _E_HALF
  sub = jnp.where(c == 0, _NU, 0)

  def chunk(g, _):
    m = g * 16 + s

    @pl.when(m < _NCHUNK)
    def _():
      eoff = half + m * 128
      pltpu.sync_copy(row_hbm.at[pl.ds(eoff, 128)], rbuf)
      pltpu.sync_copy(col_hbm.at[pl.ds(eoff, 128)], cbuf)

      def grp(k, _):
        o = k * 16
        rv = rbuf[pl.ds(o, 16)]
        rpbuf[pl.ds(o, 16)] = jnp.where(rv >= _NU, rv + _ITEM_SHIFT, rv)
        clbuf[pl.ds(o, 16)] = cbuf[pl.ds(o, 16)] - jnp.full((16,), sub,
                                                            jnp.int32)
        return 0
      lax.fori_loop(0, 8, grp, 0)
      pltpu.sync_copy(rpbuf, rowp_hbm.at[pl.ds(eoff, 128)])
      pltpu.sync_copy(clbuf, coll_hbm.at[pl.ds(eoff, 128)])
    return 0
  lax.fori_loop(0, _GMAX, chunk, 0)


_CH = 64
_NCH64 = _E_HALF // _CH
_CPT = _NCH64 // 16
_CREM = _NCH64 - 16 * _CPT
_NBLK8 = (_CPT + 1 + 7) // 8
_ACC_ROWS = 30080


@functools.partial(
    pl.kernel,
    out_type=jax.ShapeDtypeStruct((_PAD_N, _D), jnp.float32),
    mesh=plsc.VectorSubcoreMesh(**_MESH),
    compiler_params=_SC_PARAMS,
    scratch_types=[
        pltpu.VMEM_SHARED((_ACC_ROWS, _D), jnp.float32),
        pltpu.VMEM((8, _CH), jnp.int32),
        pltpu.VMEM((8, _CH), jnp.int32),
        pltpu.VMEM((_CH,), jnp.int32),
        pltpu.VMEM((_CH,), jnp.int32),
        pltpu.VMEM((_CH,), jnp.int32),
        pltpu.VMEM((_CH,), jnp.int32),
        pltpu.VMEM((_CH, _D), jnp.float32),
        pltpu.VMEM((_CH, _D), jnp.float32),
        pltpu.SemaphoreType.DMA,
        pltpu.SemaphoreType.DMA,
        pltpu.SemaphoreType.DMA,
        pltpu.SemaphoreType.DMA,
    ],
)
def _propagate_sc(y_hbm, rowp_hbm, coll_hbm, a_hbm,
                  acc, ridxblk, collblk, ridxa, ridxb, lidxa, lidxb,
                  rbufa, rbufb, sema, semb, semsa, semsb):
  c, s = _worker_ids()
  zeros16 = jnp.zeros((16,), jnp.float32)

  def zb(k, _):
    rbufa[k // 4, pl.ds((k % 4) * 16, 16)] = zeros16
    return 0
  lax.fori_loop(0, _CH * 4, zb, 0)

  nzc = jnp.where(c == 0, 20480 // _CH, _ACC_ROWS // _CH)

  def za(g, _):
    t = g * 16 + s

    @pl.when(t < nzc)
    def _():
      pltpu.sync_copy(rbufa, acc.at[pl.ds(t * _CH, _CH), :])
    return 0
  lax.fori_loop(0, _ACC_ROWS // _CH // 16 + 1, za, 0)
  plsc.subcore_barrier()

  start = s * _CPT + jnp.minimum(s, _CREM)
  cnt = _CPT + jnp.where(s < _CREM, 1, 0)
  r0base = c * _NCH64 + start

  idxs = (ridxa, ridxb)
  lids = (lidxa, lidxb)
  bufs = (rbufa, rbufb)
  sems = (sema, semb)
  ssems = (semsa, semsb)

  def block(b, _):
    base_n = b * 8

    @pl.when(base_n < cnt)
    def _():
      pltpu.sync_copy(rowp_hbm.at[pl.ds(r0base + base_n, 8), :], ridxblk)
      pltpu.sync_copy(coll_hbm.at[pl.ds(r0base + base_n, 8), :], collblk)

    for k in range(8):
      n = base_n + k
      p = (k + 1) % 2

      prev_n = n - 1

      @pl.when(jnp.logical_and(prev_n >= 0, prev_n < cnt))
      def _(p=p):
        pltpu.make_async_copy(y_hbm.at[idxs[p]], bufs[p], sems[p]).wait()
        pltpu.async_copy(bufs[p], acc.at[lids[p]], ssems[p], add=True)

      @pl.when(n < cnt)
      def _(k=k):
        q = k % 2

        @pl.when(n - 2 >= 0)
        def _():
          pltpu.make_async_copy(bufs[q], acc.at[lids[q]], ssems[q]).wait()
        for o in range(4):
          idxs[q][pl.ds(o * 16, 16)] = ridxblk[k, pl.ds(o * 16, 16)]
          lids[q][pl.ds(o * 16, 16)] = collblk[k, pl.ds(o * 16, 16)]
        pltpu.async_copy(y_hbm.at[idxs[q]], bufs[q], sems[q])
    return 0
  lax.fori_loop(0, _NBLK8, block, 0)

  @pl.when(cnt >= 2)
  def _():
    pltpu.make_async_copy(bufs[0], acc.at[lids[0]], ssems[0]).wait()

  @pl.when(cnt >= 1)
  def _():
    pltpu.make_async_copy(bufs[1], acc.at[lids[1]], ssems[1]).wait()
  plsc.subcore_barrier()

  base = jnp.where(c == 0, _PAD_U, 0)

  def wb(g, _):
    t = g * 16 + s

    @pl.when(t < nzc)
    def _():
      pltpu.sync_copy(acc.at[pl.ds(t * _CH, _CH), :],
                      a_hbm.at[pl.ds(base + t * _CH, _CH), :])
    return 0
  lax.fori_loop(0, _ACC_ROWS // _CH // 16 + 1, wb, 0)


_DW = 16


@functools.partial(
    pl.kernel,
    out_type=jax.ShapeDtypeStruct((_PAD_N, _DW), jnp.float32),
    mesh=plsc.VectorSubcoreMesh(**_MESH),
    compiler_params=_SC_PARAMS,
    scratch_types=[
        pltpu.VMEM_SHARED((_ACC_ROWS, _DW), jnp.float32),
        pltpu.VMEM((8, _CH), jnp.int32),
        pltpu.VMEM((_CH,), jnp.int32),
        pltpu.VMEM((_CH,), jnp.int32),
        pltpu.VMEM((_CH, _DW), jnp.float32),
        pltpu.VMEM((_CH, _DW), jnp.float32),
        pltpu.SemaphoreType.DMA,
        pltpu.SemaphoreType.DMA,
    ],
)
def _degrees_sc(coll_hbm, d_hbm, acc, collblk, lidxa, lidxb, ones, zeros,
                semsa, semsb):
  c, s = _worker_ids()
  ones16 = jnp.ones((16,), jnp.float32)
  zeros16 = jnp.zeros((16,), jnp.float32)

  def zb(k, _):
    ones[k, pl.ds(0, 16)] = ones16
    zeros[k, pl.ds(0, 16)] = zeros16
    return 0
  lax.fori_loop(0, _CH, zb, 0)

  nzc = jnp.where(c == 0, 20480 // _CH, _ACC_ROWS // _CH)

  def za(g, _):
    t = g * 16 + s

    @pl.when(t < nzc)
    def _():
      pltpu.sync_copy(zeros, acc.at[pl.ds(t * _CH, _CH), :])
    return 0
  lax.fori_loop(0, _ACC_ROWS // _CH // 16 + 1, za, 0)
  plsc.subcore_barrier()

  start = s * _CPT + jnp.minimum(s, _CREM)
  cnt = _CPT + jnp.where(s < _CREM, 1, 0)
  r0base = c * _NCH64 + start

  lids = (lidxa, lidxb)
  ssems = (semsa, semsb)

  def block(b, _):
    base_n = b * 8

    @pl.when(base_n < cnt)
    def _():
      pltpu.sync_copy(coll_hbm.at[pl.ds(r0base + base_n, 8), :], collblk)

    for k in range(8):
      n = base_n + k
      q = k % 2

      @pl.when(n < cnt)
      def _(k=k, q=q):
        @pl.when(n - 2 >= 0)
        def _():
          pltpu.make_async_copy(ones, acc.at[lids[q]], ssems[q]).wait()
        for o in range(4):
          lids[q][pl.ds(o * 16, 16)] = collblk[k, pl.ds(o * 16, 16)]
        pltpu.async_copy(ones, acc.at[lids[q]], ssems[q], add=True)
    return 0
  lax.fori_loop(0, _NBLK8, block, 0)

  @pl.when(cnt >= 2)
  def _():
    pltpu.make_async_copy(ones, acc.at[lids[0]], ssems[0]).wait()

  @pl.when(cnt >= 1)
  def _():
    pltpu.make_async_copy(ones, acc.at[lids[1]], ssems[1]).wait()
  plsc.subcore_barrier()

  base = jnp.where(c == 0, _PAD_U, 0)

  def wb(g, _):
    t = g * 16 + s

    @pl.when(t < nzc)
    def _():
      pltpu.sync_copy(acc.at[pl.ds(t * _CH, _CH), :],
                      d_hbm.at[pl.ds(base + t * _CH, _CH), :])
    return 0
  lax.fori_loop(0, _ACC_ROWS // _CH // 16 + 1, wb, 0)


@functools.partial(
    pl.kernel,
    out_type=[
        jax.ShapeDtypeStruct((_B, _D), jnp.float32),
        jax.ShapeDtypeStruct((_B, _D), jnp.float32),
        jax.ShapeDtypeStruct((_B, _D), jnp.float32),
    ],
    mesh=plsc.VectorSubcoreMesh(**_MESH),
    compiler_params=_SC_PARAMS,
    scratch_types=[
        pltpu.VMEM((128,), jnp.int32),
        pltpu.VMEM((128,), jnp.int32),
        pltpu.VMEM((128,), jnp.int32),
        pltpu.VMEM((128, _D), jnp.float32),
        pltpu.VMEM((128, _D), jnp.float32),
        pltpu.VMEM((128, _D), jnp.float32),
        pltpu.SemaphoreType.DMA,
    ],
)
def _batch_gather(z_hbm, u_hbm, i_hbm, j_hbm, zu_hbm, zi_hbm, zj_hbm,
                  uix, iix, jix, zu, zi, zj, sem):
  c, s = _worker_ids()
  w = s * 2 + c
  shift = jnp.full((16,), _PAD_U, jnp.int32)

  def chunk(k, _):
    boff = w * 512 + k * 128
    pltpu.sync_copy(u_hbm.at[pl.ds(boff, 128)], uix)
    pltpu.sync_copy(i_hbm.at[pl.ds(boff, 128)], iix)
    pltpu.sync_copy(j_hbm.at[pl.ds(boff, 128)], jix)

    def off(kk, _):
      o = kk * 16
      iix[pl.ds(o, 16)] = iix[pl.ds(o, 16)] + shift
      jix[pl.ds(o, 16)] = jix[pl.ds(o, 16)] + shift
      return 0
    lax.fori_loop(0, 8, off, 0)

    pltpu.async_copy(z_hbm.at[uix], zu, sem).wait()
    pltpu.async_copy(z_hbm.at[iix], zi, sem).wait()
    pltpu.async_copy(z_hbm.at[jix], zj, sem).wait()

    pltpu.sync_copy(zu, zu_hbm.at[pl.ds(boff, 128), :])
    pltpu.sync_copy(zi, zi_hbm.at[pl.ds(boff, 128), :])
    pltpu.sync_copy(zj, zj_hbm.at[pl.ds(boff, 128), :])
    return 0
  lax.fori_loop(0, 4, chunk, 0)


_ROWS_BLK = 1024
_NBLK = _PAD_N // _ROWS_BLK


def _scale(x, deg2, power):
  def body(x_ref, d_ref, o_ref):
    d = d_ref[...]
    if power == -0.5:
      sc = lax.rsqrt(d)
    else:
      sc = 1.0 / d
    sc = jnp.where(d > 0, sc, 0.0)
    o_ref[...] = x_ref[...] * sc

  return pl.pallas_call(
      body,
      grid=(_NBLK,),
      in_specs=[
          pl.BlockSpec((_ROWS_BLK, _D), lambda g: (g, 0)),
          pl.BlockSpec((_ROWS_BLK, 1), lambda g: (g, 0)),
      ],
      out_specs=pl.BlockSpec((_ROWS_BLK, _D), lambda g: (g, 0)),
      out_shape=jax.ShapeDtypeStruct((_PAD_N, _D), jnp.float32),
  )(x, deg2)


def _fuse_dense(ego_p, ego_n, a1, a2, a3, deg2,
                w0t, b0, w1t, b1, wat, ba, wqt):
  def body(ep_ref, en_ref, a1_ref, a2_ref, a3_ref, d_ref,
           w0_ref, b0_ref, w1_ref, b1_ref, wa_ref, ba_ref, wq_ref, o_ref):
    d = d_ref[...]
    dinv = jnp.where(d > 0, lax.rsqrt(d), 0.0)
    zp = (ep_ref[...] + dinv * (a1_ref[...] + a2_ref[...] + a3_ref[...])) * 0.25
    h = jnp.maximum(
        jnp.dot(en_ref[...], w0_ref[...],
                preferred_element_type=jnp.float32) + b0_ref[...], 0.0)
    zn = jnp.maximum(
        jnp.dot(h, w1_ref[...],
                preferred_element_type=jnp.float32) + b1_ref[...], 0.0)
    hp = jnp.tanh(jnp.dot(zp, wa_ref[...],
                          preferred_element_type=jnp.float32) + ba_ref[...])
    hn = jnp.tanh(jnp.dot(zn, wa_ref[...],
                          preferred_element_type=jnp.float32) + ba_ref[...])
    wp = jnp.dot(hp, wq_ref[...], preferred_element_type=jnp.float32)
    wn = jnp.dot(hn, wq_ref[...], preferred_element_type=jnp.float32)
    mx = jnp.maximum(wp, wn)
    e_p = jnp.exp(wp - mx)
    e_n = jnp.exp(wn - mx)
    ap = e_p / (e_p + e_n)
    o_ref[...] = ap * zp + (1.0 - ap) * zn

  blk = lambda g: (g, 0)
  zero = lambda g: (0, 0)
  return pl.pallas_call(
      body,
      grid=(_NBLK,),
      in_specs=[
          pl.BlockSpec((_ROWS_BLK, _D), blk),
          pl.BlockSpec((_ROWS_BLK, _D), blk),
          pl.BlockSpec((_ROWS_BLK, _D), blk),
          pl.BlockSpec((_ROWS_BLK, _D), blk),
          pl.BlockSpec((_ROWS_BLK, _D), blk),
          pl.BlockSpec((_ROWS_BLK, 1), blk),
          pl.BlockSpec((_D, _D), zero),
          pl.BlockSpec((1, _D), zero),
          pl.BlockSpec((_D, _D), zero),
          pl.BlockSpec((1, _D), zero),
          pl.BlockSpec((_D, _D), zero),
          pl.BlockSpec((1, _D), zero),
          pl.BlockSpec((_D, 1), zero),
      ],
      out_specs=pl.BlockSpec((_ROWS_BLK, _D), blk),
      out_shape=jax.ShapeDtypeStruct((_PAD_N, _D), jnp.float32),
  )(ego_p, ego_n, a1, a2, a3, deg2, w0t, b0, w1t, b1, wat, ba, wqt)


def _loss(zu, zi, zj, sgn2):
  nblk = 16
  rows = _B // nblk

  def body(u_ref, i_ref, j_ref, s_ref, o_ref):
    g = pl.program_id(0)
    u = u_ref[...]
    i = i_ref[...]
    j = j_ref[...]
    pos = jnp.sum(u * i, axis=1, keepdims=True)
    neg = jnp.sum(u * j, axis=1, keepdims=True)
    regs = jnp.sum(u * u + i * i + j * j, axis=1, keepdims=True)
    sc = s_ref[...] * pos - neg
    sig = 1.0 / (1.0 + jnp.exp(-sc))
    l = jnp.log(_GAMMA + sig)
    part = (-jnp.sum(l) + _REG * jnp.sum(regs)) * (1.0 / _B)

    @pl.when(g == 0)
    def _():
      o_ref[...] = jnp.zeros((1, 1), jnp.float32)
    o_ref[...] = o_ref[...] + jnp.reshape(part, (1, 1))

  blk = lambda g: (g, 0)
  return pl.pallas_call(
      body,
      grid=(nblk,),
      in_specs=[
          pl.BlockSpec((rows, _D), blk),
          pl.BlockSpec((rows, _D), blk),
          pl.BlockSpec((rows, _D), blk),
          pl.BlockSpec((rows, 1), blk),
      ],
      out_specs=pl.BlockSpec((1, 1), lambda g: (0, 0)),
      out_shape=jax.ShapeDtypeStruct((1, 1), jnp.float32),
  )(zu, zi, zj, sgn2)


def kernel(u, i, j, sgn, edge_index, emb_pos_u, emb_pos_i, emb_neg_u,
           emb_neg_i, W_mlp0, b_mlp0, W_mlp1, b_mlp1, W_attn, b_attn, W_q):
  row = edge_index[0].astype(jnp.int32)
  col = edge_index[1].astype(jnp.int32)

  ego_p = jnp.zeros((_PAD_N, _D), jnp.float32)
  ego_p = ego_p.at[:_NU].set(emb_pos_u).at[_PAD_U:_PAD_U + _NV].set(emb_pos_i)
  ego_n = jnp.zeros((_PAD_N, _D), jnp.float32)
  ego_n = ego_n.at[:_NU].set(emb_neg_u).at[_PAD_U:_PAD_U + _NV].set(emb_neg_i)

  rowp, coll = _remap(row, col)
  rowp2 = rowp.reshape(_EROWS, 64)
  coll2 = coll.reshape(_EROWS, 64)
  deg16 = _degrees_sc(coll2)
  deg2 = deg16[:, 0:1]

  y0 = _scale(ego_p, deg2, -0.5)
  a1 = _propagate_sc(y0, rowp2, coll2)
  y1 = _scale(a1, deg2, -1.0)
  a2 = _propagate_sc(y1, rowp2, coll2)
  y2 = _scale(a2, deg2, -1.0)
  a3 = _propagate_sc(y2, rowp2, coll2)

  z = _fuse_dense(ego_p, ego_n, a1, a2, a3, deg2,
                  W_mlp0.T, b_mlp0.reshape(1, _D),
                  W_mlp1.T, b_mlp1.reshape(1, _D),
                  W_attn.T, b_attn.reshape(1, _D),
                  W_q.reshape(1, _D).T)

  zu, zi, zj = _batch_gather(z, u.astype(jnp.int32),
                             i.astype(jnp.int32), j.astype(jnp.int32))

  out = _loss(zu, zi, zj, sgn.reshape(_B, 1))
  return out.reshape(())


# speedup vs baseline: 14.2038x; 1.1495x over previous
import functools

import jax
import jax.numpy as jnp
from jax import lax
from jax.experimental import pallas as pl
from jax.experimental.pallas import tpu as pltpu
from jax.experimental.pallas import tpu_sc as plsc

_NU = 30000
_NV = 20000
_N = 50000
_D = 64
_ITEM_OFF = 30080
_ITEM_SHIFT = _ITEM_OFF - _NU
_PAD_N = 50176
_E_HALF = 400000
_B = 16384
_REG = 0.05
_GAMMA = 1e-10

_EROWS = 12500

_MESH = dict(core_axis_name="c", subcore_axis_name="s", num_cores=2,
             num_subcores=16)
_SC_PARAMS = pltpu.CompilerParams(use_tc_tiling_on_sc=False)


def _worker_ids():
  c = lax.axis_index("c")
  s = lax.axis_index("s")
  return c, s


_CH = 64
_NCH64 = _E_HALF // _CH
_CPT = _NCH64 // 16
_CREM = _NCH64 - 16 * _CPT
_NBLK8 = (_CPT + 1 + 7) // 8
_ACC_ROWS = 30080


@functools.partial(
    pl.kernel,
    out_type=jax.ShapeDtypeStruct((_PAD_N, _D), jnp.float32),
    mesh=plsc.VectorSubcoreMesh(**_MESH),
    compiler_params=_SC_PARAMS,
    scratch_types=[
        pltpu.VMEM_SHARED((_ACC_ROWS, _D), jnp.float32),
        pltpu.VMEM((8, _CH), jnp.int32),
        pltpu.VMEM((8, _CH), jnp.int32),
        pltpu.VMEM((_CH,), jnp.int32),
        pltpu.VMEM((_CH,), jnp.int32),
        pltpu.VMEM((_CH,), jnp.int32),
        pltpu.VMEM((_CH,), jnp.int32),
        pltpu.VMEM((_CH, _D), jnp.float32),
        pltpu.VMEM((_CH, _D), jnp.float32),
        pltpu.SemaphoreType.DMA,
        pltpu.SemaphoreType.DMA,
        pltpu.SemaphoreType.DMA,
        pltpu.SemaphoreType.DMA,
    ],
)
def _propagate_sc(y_hbm, rowp_hbm, coll_hbm, a_hbm,
                  acc, ridxblk, collblk, ridxa, ridxb, lidxa, lidxb,
                  rbufa, rbufb, sema, semb, semsa, semsb):
  c, s = _worker_ids()
  zeros16 = jnp.zeros((16,), jnp.float32)
  radj = jnp.full((16,), jnp.where(c == 1, _ITEM_SHIFT, 0), jnp.int32)
  cadj = jnp.full((16,), jnp.where(c == 0, _NU, 0), jnp.int32)

  def zb(k, _):
    rbufa[k // 4, pl.ds((k % 4) * 16, 16)] = zeros16
    return 0
  lax.fori_loop(0, _CH * 4, zb, 0)

  nzc = jnp.where(c == 0, 20032 // _CH, _ACC_ROWS // _CH)

  def za(g, _):
    t = g * 16 + s

    @pl.when(t < nzc)
    def _():
      pltpu.sync_copy(rbufa, acc.at[pl.ds(t * _CH, _CH), :])
    return 0
  lax.fori_loop(0, _ACC_ROWS // _CH // 16 + 1, za, 0)
  plsc.subcore_barrier()

  start = s * _CPT + jnp.minimum(s, _CREM)
  cnt = _CPT + jnp.where(s < _CREM, 1, 0)
  r0base = c * _NCH64 + start

  idxs = (ridxa, ridxb)
  lids = (lidxa, lidxb)
  bufs = (rbufa, rbufb)
  sems = (sema, semb)
  ssems = (semsa, semsb)

  def block(b, _):
    base_n = b * 8

    @pl.when(base_n < cnt)
    def _():
      pltpu.sync_copy(rowp_hbm.at[pl.ds(r0base + base_n, 8), :], ridxblk)
      pltpu.sync_copy(coll_hbm.at[pl.ds(r0base + base_n, 8), :], collblk)

    for k in range(8):
      n = base_n + k
      p = (k + 1) % 2

      prev_n = n - 1

      @pl.when(jnp.logical_and(prev_n >= 0, prev_n < cnt))
      def _(p=p):
        pltpu.make_async_copy(y_hbm.at[idxs[p]], bufs[p], sems[p]).wait()
        pltpu.async_copy(bufs[p], acc.at[lids[p]], ssems[p], add=True)

      @pl.when(n < cnt)
      def _(k=k):
        q = k % 2

        @pl.when(n - 2 >= 0)
        def _():
          pltpu.make_async_copy(bufs[q], acc.at[lids[q]], ssems[q]).wait()
        for o in range(4):
          idxs[q][pl.ds(o * 16, 16)] = ridxblk[k, pl.ds(o * 16, 16)] + radj
          lids[q][pl.ds(o * 16, 16)] = collblk[k, pl.ds(o * 16, 16)] - cadj
        pltpu.async_copy(y_hbm.at[idxs[q]], bufs[q], sems[q])
    return 0
  lax.fori_loop(0, _NBLK8, block, 0)

  @pl.when(cnt >= 2)
  def _():
    pltpu.make_async_copy(bufs[0], acc.at[lids[0]], ssems[0]).wait()

  @pl.when(cnt >= 1)
  def _():
    pltpu.make_async_copy(bufs[1], acc.at[lids[1]], ssems[1]).wait()
  plsc.subcore_barrier()

  base = jnp.where(c == 0, _ITEM_OFF, 0)

  def wb(g, _):
    t = g * 16 + s

    @pl.when(t < nzc)
    def _():
      pltpu.sync_copy(acc.at[pl.ds(t * _CH, _CH), :],
                      a_hbm.at[pl.ds(base + t * _CH, _CH), :])
    return 0
  lax.fori_loop(0, _ACC_ROWS // _CH // 16 + 1, wb, 0)


_DW = 16


@functools.partial(
    pl.kernel,
    out_type=jax.ShapeDtypeStruct((_PAD_N, _DW), jnp.float32),
    mesh=plsc.VectorSubcoreMesh(**_MESH),
    compiler_params=_SC_PARAMS,
    scratch_types=[
        pltpu.VMEM_SHARED((_ACC_ROWS, _DW), jnp.float32),
        pltpu.VMEM((8, _CH), jnp.int32),
        pltpu.VMEM((_CH,), jnp.int32),
        pltpu.VMEM((_CH,), jnp.int32),
        pltpu.VMEM((_CH, _DW), jnp.float32),
        pltpu.VMEM((_CH, _DW), jnp.float32),
        pltpu.SemaphoreType.DMA,
        pltpu.SemaphoreType.DMA,
    ],
)
def _degrees_sc(coll_hbm, d_hbm, acc, collblk, lidxa, lidxb, ones, zeros,
                semsa, semsb):
  c, s = _worker_ids()
  ones16 = jnp.ones((16,), jnp.float32)
  zeros16 = jnp.zeros((16,), jnp.float32)
  cadj = jnp.full((16,), jnp.where(c == 0, _NU, 0), jnp.int32)

  def zb(k, _):
    ones[k, pl.ds(0, 16)] = ones16
    zeros[k, pl.ds(0, 16)] = zeros16
    return 0
  lax.fori_loop(0, _CH, zb, 0)

  nzc = jnp.where(c == 0, 20032 // _CH, _ACC_ROWS // _CH)

  def za(g, _):
    t = g * 16 + s

    @pl.when(t < nzc)
    def _():
      pltpu.sync_copy(zeros, acc.at[pl.ds(t * _CH, _CH), :])
    return 0
  lax.fori_loop(0, _ACC_ROWS // _CH // 16 + 1, za, 0)
  plsc.subcore_barrier()

  start = s * _CPT + jnp.minimum(s, _CREM)
  cnt = _CPT + jnp.where(s < _CREM, 1, 0)
  r0base = c * _NCH64 + start

  lids = (lidxa, lidxb)
  ssems = (semsa, semsb)

  def block(b, _):
    base_n = b * 8

    @pl.when(base_n < cnt)
    def _():
      pltpu.sync_copy(coll_hbm.at[pl.ds(r0base + base_n, 8), :], collblk)

    for k in range(8):
      n = base_n + k
      q = k % 2

      @pl.when(n < cnt)
      def _(k=k, q=q):
        @pl.when(n - 2 >= 0)
        def _():
          pltpu.make_async_copy(ones, acc.at[lids[q]], ssems[q]).wait()
        for o in range(4):
          lids[q][pl.ds(o * 16, 16)] = collblk[k, pl.ds(o * 16, 16)] - cadj
        pltpu.async_copy(ones, acc.at[lids[q]], ssems[q], add=True)
    return 0
  lax.fori_loop(0, _NBLK8, block, 0)

  @pl.when(cnt >= 2)
  def _():
    pltpu.make_async_copy(ones, acc.at[lids[0]], ssems[0]).wait()

  @pl.when(cnt >= 1)
  def _():
    pltpu.make_async_copy(ones, acc.at[lids[1]], ssems[1]).wait()
  plsc.subcore_barrier()

  base = jnp.where(c == 0, _ITEM_OFF, 0)

  def wb(g, _):
    t = g * 16 + s

    @pl.when(t < nzc)
    def _():
      pltpu.sync_copy(acc.at[pl.ds(t * _CH, _CH), :],
                      d_hbm.at[pl.ds(base + t * _CH, _CH), :])
    return 0
  lax.fori_loop(0, _ACC_ROWS // _CH // 16 + 1, wb, 0)


@functools.partial(
    pl.kernel,
    out_type=[
        jax.ShapeDtypeStruct((_B, _D), jnp.float32),
        jax.ShapeDtypeStruct((_B, _D), jnp.float32),
        jax.ShapeDtypeStruct((_B, _D), jnp.float32),
    ],
    mesh=plsc.VectorSubcoreMesh(**_MESH),
    compiler_params=_SC_PARAMS,
    scratch_types=[
        pltpu.VMEM((128,), jnp.int32),
        pltpu.VMEM((128,), jnp.int32),
        pltpu.VMEM((128,), jnp.int32),
        pltpu.VMEM((128, _D), jnp.float32),
        pltpu.VMEM((128, _D), jnp.float32),
        pltpu.VMEM((128, _D), jnp.float32),
        pltpu.SemaphoreType.DMA,
    ],
)
def _batch_gather(z_hbm, u_hbm, i_hbm, j_hbm, zu_hbm, zi_hbm, zj_hbm,
                  uix, iix, jix, zu, zi, zj, sem):
  c, s = _worker_ids()
  w = s * 2 + c
  shift = jnp.full((16,), _ITEM_OFF, jnp.int32)

  def chunk(k, _):
    boff = w * 512 + k * 128
    pltpu.sync_copy(u_hbm.at[pl.ds(boff, 128)], uix)
    pltpu.sync_copy(i_hbm.at[pl.ds(boff, 128)], iix)
    pltpu.sync_copy(j_hbm.at[pl.ds(boff, 128)], jix)

    def off(kk, _):
      o = kk * 16
      iix[pl.ds(o, 16)] = iix[pl.ds(o, 16)] + shift
      jix[pl.ds(o, 16)] = jix[pl.ds(o, 16)] + shift
      return 0
    lax.fori_loop(0, 8, off, 0)

    pltpu.async_copy(z_hbm.at[uix], zu, sem).wait()
    pltpu.async_copy(z_hbm.at[iix], zi, sem).wait()
    pltpu.async_copy(z_hbm.at[jix], zj, sem).wait()

    pltpu.sync_copy(zu, zu_hbm.at[pl.ds(boff, 128), :])
    pltpu.sync_copy(zi, zi_hbm.at[pl.ds(boff, 128), :])
    pltpu.sync_copy(zj, zj_hbm.at[pl.ds(boff, 128), :])
    return 0
  lax.fori_loop(0, 4, chunk, 0)


_ROWS_BLK = 1024
_NBLK = _PAD_N // _ROWS_BLK


def _scale(x, deg2, power):
  def body(x_ref, d_ref, o_ref):
    d = d_ref[...]
    if power == -0.5:
      sc = lax.rsqrt(d)
    else:
      sc = 1.0 / d
    sc = jnp.where(d > 0, sc, 0.0)
    o_ref[...] = x_ref[...] * sc

  return pl.pallas_call(
      body,
      grid=(_NBLK,),
      in_specs=[
          pl.BlockSpec((_ROWS_BLK, _D), lambda g: (g, 0)),
          pl.BlockSpec((_ROWS_BLK, 1), lambda g: (g, 0)),
      ],
      out_specs=pl.BlockSpec((_ROWS_BLK, _D), lambda g: (g, 0)),
      out_shape=jax.ShapeDtypeStruct((_PAD_N, _D), jnp.float32),
  )(x, deg2)


def _fuse_dense(ego_p, ego_n, a1, a2, a3, deg2,
                w0t, b0, w1t, b1, wat, ba, wqt):
  def body(ep_ref, en_ref, a1_ref, a2_ref, a3_ref, d_ref,
           w0_ref, b0_ref, w1_ref, b1_ref, wa_ref, ba_ref, wq_ref, o_ref):
    d = d_ref[...]
    dinv = jnp.where(d > 0, lax.rsqrt(d), 0.0)
    zp = (ep_ref[...] + dinv * (a1_ref[...] + a2_ref[...] + a3_ref[...])) * 0.25
    h = jnp.maximum(
        jnp.dot(en_ref[...], w0_ref[...],
                preferred_element_type=jnp.float32) + b0_ref[...], 0.0)
    zn = jnp.maximum(
        jnp.dot(h, w1_ref[...],
                preferred_element_type=jnp.float32) + b1_ref[...], 0.0)
    hp = jnp.tanh(jnp.dot(zp, wa_ref[...],
                          preferred_element_type=jnp.float32) + ba_ref[...])
    hn = jnp.tanh(jnp.dot(zn, wa_ref[...],
                          preferred_element_type=jnp.float32) + ba_ref[...])
    wp = jnp.dot(hp, wq_ref[...], preferred_element_type=jnp.float32)
    wn = jnp.dot(hn, wq_ref[...], preferred_element_type=jnp.float32)
    mx = jnp.maximum(wp, wn)
    e_p = jnp.exp(wp - mx)
    e_n = jnp.exp(wn - mx)
    ap = e_p / (e_p + e_n)
    o_ref[...] = ap * zp + (1.0 - ap) * zn

  blk = lambda g: (g, 0)
  zero = lambda g: (0, 0)
  return pl.pallas_call(
      body,
      grid=(_NBLK,),
      in_specs=[
          pl.BlockSpec((_ROWS_BLK, _D), blk),
          pl.BlockSpec((_ROWS_BLK, _D), blk),
          pl.BlockSpec((_ROWS_BLK, _D), blk),
          pl.BlockSpec((_ROWS_BLK, _D), blk),
          pl.BlockSpec((_ROWS_BLK, _D), blk),
          pl.BlockSpec((_ROWS_BLK, 1), blk),
          pl.BlockSpec((_D, _D), zero),
          pl.BlockSpec((1, _D), zero),
          pl.BlockSpec((_D, _D), zero),
          pl.BlockSpec((1, _D), zero),
          pl.BlockSpec((_D, _D), zero),
          pl.BlockSpec((1, _D), zero),
          pl.BlockSpec((_D, 1), zero),
      ],
      out_specs=pl.BlockSpec((_ROWS_BLK, _D), blk),
      out_shape=jax.ShapeDtypeStruct((_PAD_N, _D), jnp.float32),
  )(ego_p, ego_n, a1, a2, a3, deg2, w0t, b0, w1t, b1, wat, ba, wqt)


def _loss(zu, zi, zj, sgn2):
  nblk = 16
  rows = _B // nblk

  def body(u_ref, i_ref, j_ref, s_ref, o_ref):
    g = pl.program_id(0)
    u = u_ref[...]
    i = i_ref[...]
    j = j_ref[...]
    pos = jnp.sum(u * i, axis=1, keepdims=True)
    neg = jnp.sum(u * j, axis=1, keepdims=True)
    regs = jnp.sum(u * u + i * i + j * j, axis=1, keepdims=True)
    sc = s_ref[...] * pos - neg
    sig = 1.0 / (1.0 + jnp.exp(-sc))
    l = jnp.log(_GAMMA + sig)
    part = (-jnp.sum(l) + _REG * jnp.sum(regs)) * (1.0 / _B)

    @pl.when(g == 0)
    def _():
      o_ref[...] = jnp.zeros((1, 1), jnp.float32)
    o_ref[...] = o_ref[...] + jnp.reshape(part, (1, 1))

  blk = lambda g: (g, 0)
  return pl.pallas_call(
      body,
      grid=(nblk,),
      in_specs=[
          pl.BlockSpec((rows, _D), blk),
          pl.BlockSpec((rows, _D), blk),
          pl.BlockSpec((rows, _D), blk),
          pl.BlockSpec((rows, 1), blk),
      ],
      out_specs=pl.BlockSpec((1, 1), lambda g: (0, 0)),
      out_shape=jax.ShapeDtypeStruct((1, 1), jnp.float32),
  )(zu, zi, zj, sgn2)


def kernel(u, i, j, sgn, edge_index, emb_pos_u, emb_pos_i, emb_neg_u,
           emb_neg_i, W_mlp0, b_mlp0, W_mlp1, b_mlp1, W_attn, b_attn, W_q):
  row = edge_index[0].astype(jnp.int32)
  col = edge_index[1].astype(jnp.int32)

  ego_p = jnp.zeros((_PAD_N, _D), jnp.float32)
  ego_p = ego_p.at[:_NU].set(emb_pos_u).at[_ITEM_OFF:_ITEM_OFF + _NV].set(emb_pos_i)
  ego_n = jnp.zeros((_PAD_N, _D), jnp.float32)
  ego_n = ego_n.at[:_NU].set(emb_neg_u).at[_ITEM_OFF:_ITEM_OFF + _NV].set(emb_neg_i)

  rowp2 = row.reshape(_EROWS, 64)
  coll2 = col.reshape(_EROWS, 64)
  deg16 = _degrees_sc(coll2)
  deg2 = deg16[:, 0:1]

  y0 = _scale(ego_p, deg2, -0.5)
  a1 = _propagate_sc(y0, rowp2, coll2)
  y1 = _scale(a1, deg2, -1.0)
  a2 = _propagate_sc(y1, rowp2, coll2)
  y2 = _scale(a2, deg2, -1.0)
  a3 = _propagate_sc(y2, rowp2, coll2)

  z = _fuse_dense(ego_p, ego_n, a1, a2, a3, deg2,
                  W_mlp0.T, b_mlp0.reshape(1, _D),
                  W_mlp1.T, b_mlp1.reshape(1, _D),
                  W_attn.T, b_attn.reshape(1, _D),
                  W_q.reshape(1, _D).T)

  zu, zi, zj = _batch_gather(z, u.astype(jnp.int32),
                             i.astype(jnp.int32), j.astype(jnp.int32))

  out = _loss(zu, zi, zj, sgn.reshape(_B, 1))
  return out.reshape(())
